# baseline (speedup 1.0000x reference)
import functools

import jax
import jax.numpy as jnp
from jax import lax
from jax.experimental import pallas as pl
from jax.experimental.pallas import tpu as pltpu
from jax.experimental.pallas import tpu_sc as plsc

HID = 128
NUM_LAYERS = 4
CUTOFF = 10.0
MAX_NB = 32


def _silu(x):
    return x * jax.nn.sigmoid(x)


def _apply(p, x):
    return x @ p['w'] + p['b']


def _layernorm(p, x):
    m = jnp.mean(x, axis=-1, keepdims=True)
    v = jnp.var(x, axis=-1, keepdims=True)
    return (x - m) / jnp.sqrt(v + 1e-5) * p['g'] + p['b']


def _fusion_body(h2_ref, h3_ref, w1a_ref, w1b_ref, b1_ref, w2_ref, b2_ref,
                 wa_ref, ba_ref, wp_ref, bp_ref, pc_ref,
                 wb1_ref, wb2_ref, bb_ref,
                 hf_ref, atom_ref, pos_ref, p1_ref, p2_ref):
    t = (h2_ref[...] @ w1a_ref[...] + h3_ref[...] @ w1b_ref[...] + b1_ref[...])
    t = _silu(t)
    hf = t @ w2_ref[...] + b2_ref[...]
    hf_ref[...] = hf
    atom_ref[...] = hf @ wa_ref[...] + ba_ref[...]
    pos_ref[...] = pc_ref[...] + hf @ wp_ref[...] + bp_ref[...]
    p1_ref[...] = hf @ wb1_ref[...] + bb_ref[...]
    p2_ref[...] = hf @ wb2_ref[...]


def _fusion_heads(h2, h3, pc16, params):
    n = h2.shape[0]
    blk = 512
    w1 = params['fusion1']['w']
    w1a, w1b = w1[:HID], w1[HID:]
    wa = jnp.pad(params['atom_head']['w'], ((0, 0), (0, 5)))
    ba = jnp.pad(params['atom_head']['b'], ((0, 5)))
    wp = jnp.pad(params['pos_head']['w'], ((0, 0), (0, 13)))
    bp = jnp.pad(params['pos_head']['b'], ((0, 13)))
    wb = params['bond_head']['w']
    wb1 = jnp.pad(wb[:HID], ((0, 0), (0, 12)))
    wb2 = jnp.pad(wb[HID:], ((0, 0), (0, 12)))
    bb = jnp.pad(params['bond_head']['b'], ((0, 12)))
    grid = (n // blk,)
    rb = lambda i: (i, 0)
    full = lambda i: (0, 0)
    hf, atom, pos, p1, p2 = pl.pallas_call(
        _fusion_body,
        grid=grid,
        in_specs=[
            pl.BlockSpec((blk, HID), rb),
            pl.BlockSpec((blk, HID), rb),
            pl.BlockSpec((HID, HID), full),
            pl.BlockSpec((HID, HID), full),
            pl.BlockSpec((1, HID), full),
            pl.BlockSpec((HID, HID), full),
            pl.BlockSpec((1, HID), full),
            pl.BlockSpec((HID, 16), full),
            pl.BlockSpec((1, 16), full),
            pl.BlockSpec((HID, 16), full),
            pl.BlockSpec((1, 16), full),
            pl.BlockSpec((blk, 16), rb),
            pl.BlockSpec((HID, 16), full),
            pl.BlockSpec((HID, 16), full),
            pl.BlockSpec((1, 16), full),
        ],
        out_specs=[
            pl.BlockSpec((blk, HID), rb),
            pl.BlockSpec((blk, 16), rb),
            pl.BlockSpec((blk, 16), rb),
            pl.BlockSpec((blk, 16), rb),
            pl.BlockSpec((blk, 16), rb),
        ],
        out_shape=[
            jax.ShapeDtypeStruct((n, HID), jnp.float32),
            jax.ShapeDtypeStruct((n, 16), jnp.float32),
            jax.ShapeDtypeStruct((n, 16), jnp.float32),
            jax.ShapeDtypeStruct((n, 16), jnp.float32),
            jax.ShapeDtypeStruct((n, 16), jnp.float32),
        ],
    )(h2, h3, w1a, w1b, params['fusion1']['b'][None],
      params['fusion2']['w'], params['fusion2']['b'][None],
      wa, ba[None], wp, bp[None], pc16,
      wb1, wb2, bb[None])
    return hf, atom, pos, p1, p2


_RB = 128
_CH = 256
_FAST = 4
_BIG = 2**30


def _radius_body(pos4_ref, posT_ref, batchr_ref, batchc_ref, idx_ref, em_ref,
                 comp_ref, cmap_ref, full_ref):
    NPAD = posT_ref.shape[1]
    NCH = NPAD // _CH
    INF = jnp.float32(jnp.inf)
    R2 = jnp.float32(CUTOFF * CUTOFF)
    rb = pl.program_id(0)

    pos_r = pos4_ref[...]
    sq_r = jnp.sum(pos_r * pos_r, axis=1, keepdims=True)
    b_r = batchr_ref[...]
    b_lo = jnp.min(b_r)
    b_hi = jnp.max(b_r)
    rowid = rb * _RB + jax.lax.broadcasted_iota(jnp.int32, (_RB, 1), 0)

    bc_row = batchc_ref[...]
    actives = []
    for j in range(NCH):
        bcj = bc_row[:, j * _CH:(j + 1) * _CH]
        actives.append((jnp.max(bcj) >= b_lo) & (jnp.min(bcj) <= b_hi))
    n_active = sum(a.astype(jnp.int32) for a in actives)
    fast = n_active <= _FAST

    def chunk_d2(j):
        pc = posT_ref[:, j * _CH:(j + 1) * _CH]
        sq_c = jnp.sum(pc * pc, axis=0, keepdims=True)
        dot = jnp.dot(pos_r, pc, preferred_element_type=jnp.float32)
        d2 = sq_r + sq_c - 2.0 * dot
        bcj = bc_row[:, j * _CH:(j + 1) * _CH]
        colg = j * _CH + jax.lax.broadcasted_iota(jnp.int32, (1, _CH), 1)
        ok = (b_r == bcj) & (colg != rowid) & (d2 < R2)
        return jnp.where(ok, d2, INF), colg

    @pl.when(fast)
    def _():
        comp_ref[...] = jnp.full(comp_ref.shape, INF, jnp.float32)
        cmap_ref[...] = jnp.full(cmap_ref.shape, -1, jnp.int32)

    slot = jnp.int32(0)
    for j in range(NCH):
        def do_compact(j=j, slot=slot):
            d2c, colg = chunk_d2(j)
            comp_ref[:, pl.ds(slot * _CH, _CH)] = d2c
            cmap_ref[:, pl.ds(slot * _CH, _CH)] = colg
        pl.when(fast & actives[j])(do_compact)
        slot = slot + actives[j].astype(jnp.int32)

    @pl.when(~fast)
    def _():
        full_ref[...] = jnp.full(full_ref.shape, INF, jnp.float32)

    for j in range(NCH):
        def do_full(j=j):
            d2c, _ = chunk_d2(j)
            full_ref[:, j * _CH:(j + 1) * _CH] = d2c
        pl.when((~fast) & actives[j])(do_full)

    colsel_iota = jax.lax.broadcasted_iota(jnp.int32, (1, MAX_NB), 1)

    def extract(ref, colmap):
        def body(j, carry):
            idxacc, emacc = carry
            d2v = ref[...]
            m = jnp.min(d2v, axis=1, keepdims=True)
            valid = m < INF
            sel = d2v == m
            cand = jnp.where(sel, colmap, _BIG)
            idx = jnp.min(cand, axis=1, keepdims=True)
            idx_out = jnp.where(valid, idx, rowid)
            ref[...] = jnp.where(colmap == idx, INF, d2v)
            here = colsel_iota == j
            idxacc = jnp.where(here, idx_out, idxacc)
            emacc = jnp.where(here, valid.astype(jnp.float32), emacc)
            return idxacc, emacc
        return jax.lax.fori_loop(
            0, MAX_NB, body,
            (jnp.zeros((_RB, MAX_NB), jnp.int32),
             jnp.zeros((_RB, MAX_NB), jnp.float32)))

    @pl.when(fast)
    def _():
        ia, ea = extract(comp_ref, cmap_ref[...])
        idx_ref[...] = ia
        em_ref[...] = ea

    @pl.when(~fast)
    def _():
        colmap_full = jax.lax.broadcasted_iota(jnp.int32, (1, NPAD), 1)
        ia, ea = extract(full_ref, colmap_full)
        idx_ref[...] = ia
        em_ref[...] = ea


def _radius_graph_pallas(pos, batch, NPAD):
    N = pos.shape[0]
    pos4 = jnp.pad(pos, ((0, NPAD - N), (0, 1)))
    posT = pos4.T
    batchf = jnp.pad(batch.astype(jnp.float32), (0, NPAD - N),
                     constant_values=99.0)
    batchc = batchf[None, :]
    batchr = batchf[:, None]
    grid = (NPAD // _RB,)
    idx, em = pl.pallas_call(
        _radius_body,
        grid=grid,
        in_specs=[
            pl.BlockSpec((_RB, 4), lambda i: (i, 0)),
            pl.BlockSpec((4, NPAD), lambda i: (0, 0)),
            pl.BlockSpec((_RB, 1), lambda i: (i, 0)),
            pl.BlockSpec((1, NPAD), lambda i: (0, 0)),
        ],
        out_specs=[
            pl.BlockSpec((_RB, MAX_NB), lambda i: (i, 0)),
            pl.BlockSpec((_RB, MAX_NB), lambda i: (i, 0)),
        ],
        out_shape=[
            jax.ShapeDtypeStruct((NPAD, MAX_NB), jnp.int32),
            jax.ShapeDtypeStruct((NPAD, MAX_NB), jnp.float32),
        ],
        scratch_shapes=[
            pltpu.VMEM((_RB, _FAST * _CH), jnp.float32),
            pltpu.VMEM((1, _FAST * _CH), jnp.int32),
            pltpu.VMEM((_RB, NPAD), jnp.float32),
        ],
    )(pos4, posT, batchr, batchc)
    return idx, em


_SC_MESH = dict(core_axis_name="c", subcore_axis_name="s")
_SC_PARAMS = pltpu.CompilerParams(use_tc_tiling_on_sc=False)
_C = 128


def _sc_gather(table, idx):
    M = idx.shape[0]
    D = table.shape[1]
    per_w = M // 32
    n_it = per_w // _C

    @functools.partial(
        pl.kernel, mesh=plsc.VectorSubcoreMesh(**_SC_MESH),
        compiler_params=_SC_PARAMS,
        out_type=jax.ShapeDtypeStruct((M, D), jnp.float32),
        scratch_types=[pltpu.VMEM((_C,), jnp.int32),
                       pltpu.VMEM((_C,), jnp.int32),
                       pltpu.VMEM((_C, D), jnp.float32),
                       pltpu.VMEM((_C, D), jnp.float32),
                       pltpu.SemaphoreType.DMA,
                       pltpu.SemaphoreType.DMA],
    )
    def k(table_hbm, idx_hbm, out_hbm, i0, i1, r0, r1, s0, s1):
        wid = lax.axis_index("s") * 2 + lax.axis_index("c")
        base = wid * per_w

        def off(i):
            return base + jnp.minimum(i, n_it - 1) * _C

        pltpu.sync_copy(idx_hbm.at[pl.ds(base, _C)], i0)
        pltpu.async_copy(table_hbm.at[i0], r0, s0)
        pltpu.sync_copy(idx_hbm.at[pl.ds(base + _C, _C)], i1)
        pltpu.async_copy(table_hbm.at[i1], r1, s1)

        def body(g, carry):
            i = 2 * g
            for iv, rv, sv, ch in ((i0, r0, s0, i), (i1, r1, s1, i + 1)):
                pltpu.make_async_copy(table_hbm.at[iv], rv, sv).wait()
                pltpu.sync_copy(rv, out_hbm.at[pl.ds(base + ch * _C, _C)])
                pltpu.sync_copy(idx_hbm.at[pl.ds(off(ch + 2), _C)], iv)
                pltpu.async_copy(table_hbm.at[iv], rv, sv)
            return carry

        lax.fori_loop(0, n_it // 2, body, 0)
        pltpu.make_async_copy(table_hbm.at[i0], r0, s0).wait()
        pltpu.make_async_copy(table_hbm.at[i1], r1, s1).wait()

    return k(table, idx)


def _sc_gather_scatter(table, gidx, sidx, tp):
    M = gidx.shape[0]
    D = table.shape[1]
    DH = D // 2
    per_w = M // 32
    n_it = per_w // _C
    rpt = tp // 16

    @functools.partial(
        pl.kernel, mesh=plsc.VectorSubcoreMesh(**_SC_MESH),
        compiler_params=_SC_PARAMS,
        out_type=[jax.ShapeDtypeStruct((2 * tp, DH), jnp.float32),
                  jax.ShapeDtypeStruct((2 * tp, DH), jnp.float32)],
        scratch_types=[pltpu.VMEM((_C,), jnp.int32),
                       pltpu.VMEM((_C,), jnp.int32),
                       pltpu.VMEM((_C,), jnp.int32),
                       pltpu.VMEM((_C,), jnp.int32),
                       pltpu.VMEM((_C, DH), jnp.float32),
                       pltpu.VMEM((_C, DH), jnp.float32),
                       pltpu.VMEM((rpt, DH), jnp.float32),
                       pltpu.VMEM_SHARED((tp, DH), jnp.float32),
                       pltpu.SemaphoreType.DMA,
                       pltpu.SemaphoreType.DMA],
    )
    def k(t0_hbm, t1_hbm, gidx_hbm, sidx_hbm, zero_hbm, o0_hbm, o1_hbm,
          g0, g1, s0i, s1i, r0, r1, stage_v, acc_sh, sm0, sm1):
        c = lax.axis_index("c")
        s = lax.axis_index("s")
        base = (c * 16 + s) * per_w

        def off(i):
            return base + jnp.minimum(i, n_it - 1) * _C

        for th, oh in ((t0_hbm, o0_hbm), (t1_hbm, o1_hbm)):
            pltpu.sync_copy(zero_hbm.at[pl.ds(s * rpt, rpt)], stage_v)
            pltpu.sync_copy(stage_v, acc_sh.at[pl.ds(s * rpt, rpt)])
            plsc.subcore_barrier()

            pltpu.sync_copy(gidx_hbm.at[pl.ds(base, _C)], g0)
            pltpu.sync_copy(sidx_hbm.at[pl.ds(base, _C)], s0i)
            pltpu.async_copy(th.at[g0], r0, sm0)
            pltpu.sync_copy(gidx_hbm.at[pl.ds(base + _C, _C)], g1)
            pltpu.sync_copy(sidx_hbm.at[pl.ds(base + _C, _C)], s1i)
            pltpu.async_copy(th.at[g1], r1, sm1)

            def body(g, carry, th=th):
                i = 2 * g
                for gv, sv, rv, sm, ch in ((g0, s0i, r0, sm0, i),
                                           (g1, s1i, r1, sm1, i + 1)):
                    pltpu.make_async_copy(th.at[gv], rv, sm).wait()
                    pltpu.sync_copy(rv, acc_sh.at[sv], add=True)
                    pltpu.sync_copy(gidx_hbm.at[pl.ds(off(ch + 2), _C)], gv)
                    pltpu.sync_copy(sidx_hbm.at[pl.ds(off(ch + 2), _C)], sv)
                    pltpu.async_copy(th.at[gv], rv, sm)
                return carry

            lax.fori_loop(0, n_it // 2, body, 0)
            pltpu.make_async_copy(th.at[g0], r0, sm0).wait()
            pltpu.make_async_copy(th.at[g1], r1, sm1).wait()
            plsc.subcore_barrier()
            pltpu.sync_copy(acc_sh.at[pl.ds(s * rpt, rpt)], stage_v)
            pltpu.sync_copy(stage_v, oh.at[pl.ds(c * tp + s * rpt, rpt)])

    o0, o1 = k(table[:, :DH], table[:, DH:], gidx, sidx,
               jnp.zeros((tp, DH), jnp.float32))
    return jnp.concatenate([o0, o1], axis=1)


def _sc_scatter_add(values, sidx, tp):
    M = values.shape[0]
    D = values.shape[1]
    per_w = M // 32
    n_it = per_w // _C
    rpt = tp // 16

    @functools.partial(
        pl.kernel, mesh=plsc.VectorSubcoreMesh(**_SC_MESH),
        compiler_params=_SC_PARAMS,
        out_type=jax.ShapeDtypeStruct((2 * tp, D), jnp.float32),
        scratch_types=[pltpu.VMEM((_C,), jnp.int32),
                       pltpu.VMEM((_C, D), jnp.float32),
                       pltpu.VMEM((rpt, D), jnp.float32),
                       pltpu.VMEM_SHARED((tp, D), jnp.float32)],
    )
    def k(val_hbm, sidx_hbm, zero_hbm, out_hbm,
          si_v, rows_v, stage_v, acc_sh):
        c = lax.axis_index("c")
        s = lax.axis_index("s")
        pltpu.sync_copy(zero_hbm.at[pl.ds(s * rpt, rpt)], stage_v)
        pltpu.sync_copy(stage_v, acc_sh.at[pl.ds(s * rpt, rpt)])
        plsc.subcore_barrier()
        base = (c * 16 + s) * per_w

        def body(i, carry):
            off = base + i * _C
            pltpu.sync_copy(sidx_hbm.at[pl.ds(off, _C)], si_v)
            pltpu.sync_copy(val_hbm.at[pl.ds(off, _C)], rows_v)
            pltpu.sync_copy(rows_v, acc_sh.at[si_v], add=True)
            return carry

        lax.fori_loop(0, n_it, body, 0)
        plsc.subcore_barrier()
        pltpu.sync_copy(acc_sh.at[pl.ds(s * rpt, rpt)], stage_v)
        pltpu.sync_copy(stage_v, out_hbm.at[pl.ds(c * tp + s * rpt, rpt)])

    return k(values, sidx, jnp.zeros((tp, D), jnp.float32))


def _add_body(a_ref, b_ref, o_ref):
    o_ref[...] = a_ref[...] + b_ref[...]


def _tc_add(a, b):
    n, d = a.shape
    blk = 4096
    return pl.pallas_call(
        _add_body,
        grid=(n // blk,),
        in_specs=[pl.BlockSpec((blk, d), lambda i: (i, 0)),
                  pl.BlockSpec((blk, d), lambda i: (i, 0))],
        out_specs=pl.BlockSpec((blk, d), lambda i: (i, 0)),
        out_shape=jax.ShapeDtypeStruct((n, d), jnp.float32),
    )(a, b)


_NB = 128


def _mm_body(x_ref, w_ref, b_ref, o_ref):
    o_ref[...] = jnp.dot(x_ref[...], w_ref[...],
                         preferred_element_type=jnp.float32) + b_ref[...]


def _tc_matmul(x, w, b):
    n, kdim = x.shape
    dout = w.shape[1]
    return pl.pallas_call(
        _mm_body,
        grid=(n // _NB,),
        in_specs=[pl.BlockSpec((_NB, kdim), lambda i: (i, 0)),
                  pl.BlockSpec((kdim, dout), lambda i: (0, 0)),
                  pl.BlockSpec((1, dout), lambda i: (0, 0))],
        out_specs=pl.BlockSpec((_NB, dout), lambda i: (i, 0)),
        out_shape=jax.ShapeDtypeStruct((n, dout), jnp.float32),
    )(x, w, b[None])


def _gconv_post_body(h2_ref, xt_ref, s0_ref, s1_ref, e0_ref, e1_ref,
                     wev_ref, bev_ref, g_ref, b_ref, o_ref):
    sc = e0_ref[...] + e1_ref[...]
    out = (s0_ref[...] + s1_ref[...] + xt_ref[...]
           + sc[:, 0:1] * wev_ref[...] + sc[:, 1:2] * bev_ref[...])
    y = _silu(out)
    mu = jnp.mean(y, axis=1, keepdims=True)
    v = jnp.mean((y - mu) ** 2, axis=1, keepdims=True)
    o_ref[...] = h2_ref[...] + (y - mu) / jnp.sqrt(v + 1e-5) * g_ref[...] + b_ref[...]


def _tc_gconv_post(h2, xt, s0, s1, e0, e1, p):
    n = h2.shape[0]
    rb = lambda i: (i, 0)
    w0 = lambda i: (0, 0)
    return pl.pallas_call(
        _gconv_post_body,
        grid=(n // _NB,),
        in_specs=[pl.BlockSpec((_NB, HID), rb), pl.BlockSpec((_NB, HID), rb),
                  pl.BlockSpec((_NB, HID), rb), pl.BlockSpec((_NB, HID), rb),
                  pl.BlockSpec((_NB, 16), rb), pl.BlockSpec((_NB, 16), rb),
                  pl.BlockSpec((1, HID), w0), pl.BlockSpec((1, HID), w0),
                  pl.BlockSpec((1, HID), w0), pl.BlockSpec((1, HID), w0)],
        out_specs=pl.BlockSpec((_NB, HID), rb),
        out_shape=jax.ShapeDtypeStruct((n, HID), jnp.float32),
    )(h2, xt, s0, s1, e0, e1,
      p['edge']['w'], p['edge']['b'][None], p['ln']['g'][None], p['ln']['b'][None])


def _egnn_tc_body(has_att, tanh_flag,
                  h_ref, pos_ref, g_ref, em_ref,
                  w1a_ref, w1b_ref, w1r_ref, b1_ref, w2_ref, b2_ref,
                  wc1_ref, bc1_ref, wc2r_ref, bc2_ref,
                  watt_ref, batt_ref,
                  n1a_ref, n1b_ref, bn1_ref, wn2_ref, bn2_ref,
                  gln_ref, bln_ref,
                  ho_ref, po_ref):
    EB = _NB * MAX_NB
    h_r = h_ref[...]
    g = g_ref[...]
    hc = g[:, :HID]
    pc = g[:, HID:HID + 16]
    pos_r = pos_ref[...]
    pos_rep = jnp.reshape(
        jnp.broadcast_to(pos_r[:, None, :], (_NB, MAX_NB, 16)), (EB, 16))
    radial = pos_rep - pc
    rn = jnp.sqrt(jnp.sum(radial * radial, axis=1, keepdims=True))
    rn = jnp.maximum(rn, 1e-8)
    t_r = jnp.dot(h_r, w1a_ref[...], preferred_element_type=jnp.float32)
    t_rep = jnp.reshape(
        jnp.broadcast_to(t_r[:, None, :], (_NB, MAX_NB, HID)), (EB, HID))
    x1 = _silu(t_rep + jnp.dot(hc, w1b_ref[...], preferred_element_type=jnp.float32)
               + rn * w1r_ref[...] + b1_ref[...])
    m = _silu(jnp.dot(x1, w2_ref[...], preferred_element_type=jnp.float32)
              + b2_ref[...])
    cmid = _silu(jnp.dot(m, wc1_ref[...], preferred_element_type=jnp.float32)
                 + bc1_ref[...])
    cd = jnp.sum(cmid * wc2r_ref[...], axis=1, keepdims=True) + bc2_ref[:, 0:1]
    if tanh_flag:
        cd = jnp.tanh(cd)
    emv = em_ref[...][:, 0:1]
    cu = cd * (radial / rn) * emv
    cu3 = jnp.reshape(cu, (_NB, MAX_NB, 16))
    cu_sum = jnp.zeros((_NB, 16), jnp.float32)
    for kk in range(MAX_NB):
        cu_sum = cu_sum + cu3[:, kk, :]
    po_ref[...] = pos_r + cu_sum
    if has_att:
        m = m * jax.nn.sigmoid(
            jnp.sum(m * watt_ref[...], axis=1, keepdims=True) + batt_ref[:, 0:1])
    m = m * emv
    m3 = jnp.reshape(m, (_NB, MAX_NB, HID))
    agg = jnp.zeros((_NB, HID), jnp.float32)
    for kk in range(MAX_NB):
        agg = agg + m3[:, kk, :]
    hn = _silu(jnp.dot(h_r, n1a_ref[...], preferred_element_type=jnp.float32)
               + jnp.dot(agg, n1b_ref[...], preferred_element_type=jnp.float32)
               + bn1_ref[...])
    hn = jnp.dot(hn, wn2_ref[...], preferred_element_type=jnp.float32) + bn2_ref[...]
    hnew = h_r + hn
    mu = jnp.mean(hnew, axis=1, keepdims=True)
    v = jnp.mean((hnew - mu) ** 2, axis=1, keepdims=True)
    ho_ref[...] = (hnew - mu) / jnp.sqrt(v + 1e-5) * gln_ref[...] + bln_ref[...]


def _tc_egnn_layer(h, pos16, gathered, em, p, has_att, tanh_flag):
    n = h.shape[0]
    rb = lambda i: (i, 0)
    eb = lambda i: (i, 0)
    w0 = lambda i: (0, 0)
    w1 = p['e1']['w']
    if has_att:
        watt = p['att']['w'].T
        batt = jnp.broadcast_to(p['att']['b'][None], (1, HID))
    else:
        watt = jnp.zeros((1, HID), jnp.float32)
        batt = jnp.zeros((1, HID), jnp.float32)
    nw = p['n1']['w']
    ho, po = pl.pallas_call(
        functools.partial(_egnn_tc_body, has_att, tanh_flag),
        grid=(n // _NB,),
        in_specs=[pl.BlockSpec((_NB, HID), rb),
                  pl.BlockSpec((_NB, 16), rb),
                  pl.BlockSpec((_NB * MAX_NB, HID + 16), eb),
                  pl.BlockSpec((_NB * MAX_NB, 8), eb),
                  pl.BlockSpec((HID, HID), w0), pl.BlockSpec((HID, HID), w0),
                  pl.BlockSpec((1, HID), w0), pl.BlockSpec((1, HID), w0),
                  pl.BlockSpec((HID, HID), w0), pl.BlockSpec((1, HID), w0),
                  pl.BlockSpec((HID, HID), w0), pl.BlockSpec((1, HID), w0),
                  pl.BlockSpec((1, HID), w0), pl.BlockSpec((1, HID), w0),
                  pl.BlockSpec((1, HID), w0), pl.BlockSpec((1, HID), w0),
                  pl.BlockSpec((HID, HID), w0), pl.BlockSpec((HID, HID), w0),
                  pl.BlockSpec((1, HID), w0), pl.BlockSpec((HID, HID), w0),
                  pl.BlockSpec((1, HID), w0),
                  pl.BlockSpec((1, HID), w0), pl.BlockSpec((1, HID), w0)],
        out_specs=[pl.BlockSpec((_NB, HID), rb), pl.BlockSpec((_NB, 16), rb)],
        out_shape=[jax.ShapeDtypeStruct((n, HID), jnp.float32),
                   jax.ShapeDtypeStruct((n, 16), jnp.float32)],
    )(h, pos16, gathered, em,
      w1[:HID], w1[HID:2 * HID], w1[2 * HID:2 * HID + 1], p['e1']['b'][None],
      p['e2']['w'], p['e2']['b'][None],
      p['c1']['w'], p['c1']['b'][None],
      p['c2']['w'].T, jnp.broadcast_to(p['c2']['b'][None], (1, HID)),
      watt, batt,
      nw[:HID], nw[HID:], p['n1']['b'][None],
      p['n2']['w'], p['n2']['b'][None],
      p['ln']['g'][None], p['ln']['b'][None])
    return ho, po


def kernel(x, pos, edge_attr, params, edge_index, batch):
    N = x.shape[0]
    E = edge_index.shape[1]
    NPAD = ((N + _CH) // _CH) * _CH
    EPAD = ((E + 4095) // 4096) * 4096

    idxp, emp = _radius_graph_pallas(pos, batch, NPAD)
    rcol_flat = idxp.reshape(-1)
    em8 = jnp.pad(emp.reshape(-1)[:, None], ((0, 0), (0, 7)))

    xp = jnp.pad(x, ((0, NPAD - N), (0, 0)))
    pos16 = jnp.pad(pos, ((0, NPAD - N), (0, 13)))
    row2 = jnp.pad(edge_index[0], (0, EPAD - E), constant_values=NPAD - 1)
    col2 = jnp.pad(edge_index[1], (0, EPAD - E))
    ea16 = jnp.pad(
        jnp.concatenate([edge_attr, jnp.ones((E, 1), jnp.float32)], axis=1),
        ((0, EPAD - E), (0, 14)))

    h = _tc_matmul(xp, params['embed8']['w'], params['embed8']['b'])

    esum = _sc_scatter_add(ea16, row2, NPAD)
    e0, e1 = esum[:NPAD], esum[NPAD:]
    p2d = jax.tree.map(lambda *xs: jnp.stack(xs), *params['gnn2d'])

    def gstep(carry, p):
        h2 = carry
        xt = _tc_matmul(h2, p['lin']['w'], p['lin']['b'])
        s = _sc_gather_scatter(xt, col2, row2, NPAD)
        return _tc_gconv_post(h2, xt, s[:NPAD], s[NPAD:], e0, e1, p), None

    h2, _ = lax.scan(gstep, h, p2d)

    h3, pc = h, pos16
    for i, p in enumerate(params['egnn']):
        tb = jnp.concatenate([h3, pc], axis=1)
        gathered = _sc_gather(tb, rcol_flat)
        h3, pc = _tc_egnn_layer(h3, pc, gathered, em8, p,
                                has_att=('att' in p),
                                tanh_flag=(i == NUM_LAYERS - 1))

    hf, atom, pos_pred, p1, p2 = _fusion_heads(h2, h3, pc, params)
    bond = _tc_add(_sc_gather(p1, row2), _sc_gather(p2, col2))
    return atom[:N, :11], pos_pred[:N, :3], bond[:E, :4]


# speedup vs baseline: 1.0551x; 1.0551x over previous
import functools

import jax
import jax.numpy as jnp
from jax import lax
from jax.experimental import pallas as pl
from jax.experimental.pallas import tpu as pltpu
from jax.experimental.pallas import tpu_sc as plsc

HID = 128
NUM_LAYERS = 4
CUTOFF = 10.0
MAX_NB = 32


def _silu(x):
    return x * jax.nn.sigmoid(x)


def _apply(p, x):
    return x @ p['w'] + p['b']


def _layernorm(p, x):
    m = jnp.mean(x, axis=-1, keepdims=True)
    v = jnp.var(x, axis=-1, keepdims=True)
    return (x - m) / jnp.sqrt(v + 1e-5) * p['g'] + p['b']


def _fusion_body(h2_ref, h3_ref, w1a_ref, w1b_ref, b1_ref, w2_ref, b2_ref,
                 wa_ref, ba_ref, wp_ref, bp_ref, pc_ref,
                 wb1_ref, wb2_ref, bb_ref,
                 hf_ref, atom_ref, pos_ref, p1_ref, p2_ref):
    t = (h2_ref[...] @ w1a_ref[...] + h3_ref[...] @ w1b_ref[...] + b1_ref[...])
    t = _silu(t)
    hf = t @ w2_ref[...] + b2_ref[...]
    hf_ref[...] = hf
    atom_ref[...] = hf @ wa_ref[...] + ba_ref[...]
    pos_ref[...] = pc_ref[...] + hf @ wp_ref[...] + bp_ref[...]
    p1_ref[...] = hf @ wb1_ref[...] + bb_ref[...]
    p2_ref[...] = hf @ wb2_ref[...]


def _fusion_heads(h2, h3, pc16, params):
    n = h2.shape[0]
    blk = 512
    w1 = params['fusion1']['w']
    w1a, w1b = w1[:HID], w1[HID:]
    wa = jnp.pad(params['atom_head']['w'], ((0, 0), (0, 5)))
    ba = jnp.pad(params['atom_head']['b'], ((0, 5)))
    wp = jnp.pad(params['pos_head']['w'], ((0, 0), (0, 13)))
    bp = jnp.pad(params['pos_head']['b'], ((0, 13)))
    wb = params['bond_head']['w']
    wb1 = jnp.pad(wb[:HID], ((0, 0), (0, 12)))
    wb2 = jnp.pad(wb[HID:], ((0, 0), (0, 12)))
    bb = jnp.pad(params['bond_head']['b'], ((0, 12)))
    grid = (n // blk,)
    rb = lambda i: (i, 0)
    full = lambda i: (0, 0)
    hf, atom, pos, p1, p2 = pl.pallas_call(
        _fusion_body,
        grid=grid,
        in_specs=[
            pl.BlockSpec((blk, HID), rb),
            pl.BlockSpec((blk, HID), rb),
            pl.BlockSpec((HID, HID), full),
            pl.BlockSpec((HID, HID), full),
            pl.BlockSpec((1, HID), full),
            pl.BlockSpec((HID, HID), full),
            pl.BlockSpec((1, HID), full),
            pl.BlockSpec((HID, 16), full),
            pl.BlockSpec((1, 16), full),
            pl.BlockSpec((HID, 16), full),
            pl.BlockSpec((1, 16), full),
            pl.BlockSpec((blk, 16), rb),
            pl.BlockSpec((HID, 16), full),
            pl.BlockSpec((HID, 16), full),
            pl.BlockSpec((1, 16), full),
        ],
        out_specs=[
            pl.BlockSpec((blk, HID), rb),
            pl.BlockSpec((blk, 16), rb),
            pl.BlockSpec((blk, 16), rb),
            pl.BlockSpec((blk, 16), rb),
            pl.BlockSpec((blk, 16), rb),
        ],
        out_shape=[
            jax.ShapeDtypeStruct((n, HID), jnp.float32),
            jax.ShapeDtypeStruct((n, 16), jnp.float32),
            jax.ShapeDtypeStruct((n, 16), jnp.float32),
            jax.ShapeDtypeStruct((n, 16), jnp.float32),
            jax.ShapeDtypeStruct((n, 16), jnp.float32),
        ],
    )(h2, h3, w1a, w1b, params['fusion1']['b'][None],
      params['fusion2']['w'], params['fusion2']['b'][None],
      wa, ba[None], wp, bp[None], pc16,
      wb1, wb2, bb[None])
    return hf, atom, pos, p1, p2


_RB = 128
_CH = 256
_FAST = 4
_BIG = 2**30


def _radius_body(pos4_ref, posT_ref, batchr_ref, batchc_ref, idx_ref, em_ref,
                 comp_ref, cmap_ref, full_ref):
    NPAD = posT_ref.shape[1]
    NCH = NPAD // _CH
    INF = jnp.float32(jnp.inf)
    R2 = jnp.float32(CUTOFF * CUTOFF)
    rb = pl.program_id(0)

    pos_r = pos4_ref[...]
    sq_r = jnp.sum(pos_r * pos_r, axis=1, keepdims=True)
    b_r = batchr_ref[...]
    b_lo = jnp.min(b_r)
    b_hi = jnp.max(b_r)
    rowid = rb * _RB + jax.lax.broadcasted_iota(jnp.int32, (_RB, 1), 0)

    bc_row = batchc_ref[...]
    actives = []
    for j in range(NCH):
        bcj = bc_row[:, j * _CH:(j + 1) * _CH]
        actives.append((jnp.max(bcj) >= b_lo) & (jnp.min(bcj) <= b_hi))
    n_active = sum(a.astype(jnp.int32) for a in actives)
    fast = n_active <= _FAST

    def chunk_d2(j):
        pc = posT_ref[:, j * _CH:(j + 1) * _CH]
        sq_c = jnp.sum(pc * pc, axis=0, keepdims=True)
        dot = jnp.dot(pos_r, pc, preferred_element_type=jnp.float32)
        d2 = sq_r + sq_c - 2.0 * dot
        bcj = bc_row[:, j * _CH:(j + 1) * _CH]
        colg = j * _CH + jax.lax.broadcasted_iota(jnp.int32, (1, _CH), 1)
        ok = (b_r == bcj) & (colg != rowid) & (d2 < R2)
        return jnp.where(ok, d2, INF), colg

    @pl.when(fast)
    def _():
        comp_ref[...] = jnp.full(comp_ref.shape, INF, jnp.float32)
        cmap_ref[...] = jnp.full(cmap_ref.shape, -1, jnp.int32)

    slot = jnp.int32(0)
    for j in range(NCH):
        def do_compact(j=j, slot=slot):
            d2c, colg = chunk_d2(j)
            comp_ref[:, pl.ds(slot * _CH, _CH)] = d2c
            cmap_ref[:, pl.ds(slot * _CH, _CH)] = colg
        pl.when(fast & actives[j])(do_compact)
        slot = slot + actives[j].astype(jnp.int32)

    @pl.when(~fast)
    def _():
        full_ref[...] = jnp.full(full_ref.shape, INF, jnp.float32)

    for j in range(NCH):
        def do_full(j=j):
            d2c, _ = chunk_d2(j)
            full_ref[:, j * _CH:(j + 1) * _CH] = d2c
        pl.when((~fast) & actives[j])(do_full)

    colsel_iota = jax.lax.broadcasted_iota(jnp.int32, (1, MAX_NB), 1)

    def extract(ref, colmap):
        def body(j, carry):
            idxacc, emacc = carry
            d2v = ref[...]
            m = jnp.min(d2v, axis=1, keepdims=True)
            valid = m < INF
            sel = d2v == m
            cand = jnp.where(sel, colmap, _BIG)
            idx = jnp.min(cand, axis=1, keepdims=True)
            idx_out = jnp.where(valid, idx, rowid)
            ref[...] = jnp.where(colmap == idx, INF, d2v)
            here = colsel_iota == j
            idxacc = jnp.where(here, idx_out, idxacc)
            emacc = jnp.where(here, valid.astype(jnp.float32), emacc)
            return idxacc, emacc
        return jax.lax.fori_loop(
            0, MAX_NB, body,
            (jnp.zeros((_RB, MAX_NB), jnp.int32),
             jnp.zeros((_RB, MAX_NB), jnp.float32)))

    @pl.when(fast)
    def _():
        ia, ea = extract(comp_ref, cmap_ref[...])
        idx_ref[...] = ia
        em_ref[...] = ea

    @pl.when(~fast)
    def _():
        colmap_full = jax.lax.broadcasted_iota(jnp.int32, (1, NPAD), 1)
        ia, ea = extract(full_ref, colmap_full)
        idx_ref[...] = ia
        em_ref[...] = ea


def _radius_graph_pallas(pos, batch, NPAD):
    N = pos.shape[0]
    pos4 = jnp.pad(pos, ((0, NPAD - N), (0, 1)))
    posT = pos4.T
    batchf = jnp.pad(batch.astype(jnp.float32), (0, NPAD - N),
                     constant_values=99.0)
    batchc = batchf[None, :]
    batchr = batchf[:, None]
    grid = (NPAD // _RB,)
    idx, em = pl.pallas_call(
        _radius_body,
        grid=grid,
        in_specs=[
            pl.BlockSpec((_RB, 4), lambda i: (i, 0)),
            pl.BlockSpec((4, NPAD), lambda i: (0, 0)),
            pl.BlockSpec((_RB, 1), lambda i: (i, 0)),
            pl.BlockSpec((1, NPAD), lambda i: (0, 0)),
        ],
        out_specs=[
            pl.BlockSpec((_RB, MAX_NB), lambda i: (i, 0)),
            pl.BlockSpec((_RB, MAX_NB), lambda i: (i, 0)),
        ],
        out_shape=[
            jax.ShapeDtypeStruct((NPAD, MAX_NB), jnp.int32),
            jax.ShapeDtypeStruct((NPAD, MAX_NB), jnp.float32),
        ],
        scratch_shapes=[
            pltpu.VMEM((_RB, _FAST * _CH), jnp.float32),
            pltpu.VMEM((1, _FAST * _CH), jnp.int32),
            pltpu.VMEM((_RB, NPAD), jnp.float32),
        ],
    )(pos4, posT, batchr, batchc)
    return idx, em


_SC_MESH = dict(core_axis_name="c", subcore_axis_name="s")
_SC_PARAMS = pltpu.CompilerParams(use_tc_tiling_on_sc=False)
_C = 128


def _sc_gather(table, idx):
    M = idx.shape[0]
    D = table.shape[1]
    per_w = M // 32
    n_it = per_w // _C

    @functools.partial(
        pl.kernel, mesh=plsc.VectorSubcoreMesh(**_SC_MESH),
        compiler_params=_SC_PARAMS,
        out_type=jax.ShapeDtypeStruct((M, D), jnp.float32),
        scratch_types=[pltpu.VMEM((_C,), jnp.int32),
                       pltpu.VMEM((_C,), jnp.int32),
                       pltpu.VMEM((_C, D), jnp.float32),
                       pltpu.VMEM((_C, D), jnp.float32),
                       pltpu.SemaphoreType.DMA,
                       pltpu.SemaphoreType.DMA],
    )
    def k(table_hbm, idx_hbm, out_hbm, i0, i1, r0, r1, s0, s1):
        wid = lax.axis_index("s") * 2 + lax.axis_index("c")
        base = wid * per_w

        def off(i):
            return base + jnp.minimum(i, n_it - 1) * _C

        pltpu.sync_copy(idx_hbm.at[pl.ds(base, _C)], i0)
        pltpu.async_copy(table_hbm.at[i0], r0, s0)
        pltpu.sync_copy(idx_hbm.at[pl.ds(base + _C, _C)], i1)
        pltpu.async_copy(table_hbm.at[i1], r1, s1)

        def body(g, carry):
            i = 2 * g
            for iv, rv, sv, ch in ((i0, r0, s0, i), (i1, r1, s1, i + 1)):
                pltpu.make_async_copy(table_hbm.at[iv], rv, sv).wait()
                pltpu.sync_copy(rv, out_hbm.at[pl.ds(base + ch * _C, _C)])
                pltpu.sync_copy(idx_hbm.at[pl.ds(off(ch + 2), _C)], iv)
                pltpu.async_copy(table_hbm.at[iv], rv, sv)
            return carry

        lax.fori_loop(0, n_it // 2, body, 0)
        pltpu.make_async_copy(table_hbm.at[i0], r0, s0).wait()
        pltpu.make_async_copy(table_hbm.at[i1], r1, s1).wait()

    return k(table, idx)


def _sc_gather_scatter(table, gidx, sidx, tp):
    M = gidx.shape[0]
    D = table.shape[1]
    DH = D // 2
    per_w = M // 32
    n_it = per_w // _C
    rpt = tp // 16

    @functools.partial(
        pl.kernel, mesh=plsc.VectorSubcoreMesh(**_SC_MESH),
        compiler_params=_SC_PARAMS,
        out_type=[jax.ShapeDtypeStruct((2 * tp, DH), jnp.float32),
                  jax.ShapeDtypeStruct((2 * tp, DH), jnp.float32)],
        scratch_types=[pltpu.VMEM((_C,), jnp.int32),
                       pltpu.VMEM((_C,), jnp.int32),
                       pltpu.VMEM((_C,), jnp.int32),
                       pltpu.VMEM((_C,), jnp.int32),
                       pltpu.VMEM((_C, DH), jnp.float32),
                       pltpu.VMEM((_C, DH), jnp.float32),
                       pltpu.VMEM((rpt, DH), jnp.float32),
                       pltpu.VMEM_SHARED((tp, DH), jnp.float32),
                       pltpu.SemaphoreType.DMA,
                       pltpu.SemaphoreType.DMA],
    )
    def k(t0_hbm, t1_hbm, gidx_hbm, sidx_hbm, zero_hbm, o0_hbm, o1_hbm,
          g0, g1, s0i, s1i, r0, r1, stage_v, acc_sh, sm0, sm1):
        c = lax.axis_index("c")
        s = lax.axis_index("s")
        base = (c * 16 + s) * per_w

        def off(i):
            return base + jnp.minimum(i, n_it - 1) * _C

        for th, oh in ((t0_hbm, o0_hbm), (t1_hbm, o1_hbm)):
            pltpu.sync_copy(zero_hbm.at[pl.ds(s * rpt, rpt)], stage_v)
            pltpu.sync_copy(stage_v, acc_sh.at[pl.ds(s * rpt, rpt)])
            plsc.subcore_barrier()

            pltpu.sync_copy(gidx_hbm.at[pl.ds(base, _C)], g0)
            pltpu.sync_copy(sidx_hbm.at[pl.ds(base, _C)], s0i)
            pltpu.async_copy(th.at[g0], r0, sm0)
            pltpu.sync_copy(gidx_hbm.at[pl.ds(base + _C, _C)], g1)
            pltpu.sync_copy(sidx_hbm.at[pl.ds(base + _C, _C)], s1i)
            pltpu.async_copy(th.at[g1], r1, sm1)

            def body(g, carry, th=th):
                i = 2 * g
                for gv, sv, rv, sm, ch in ((g0, s0i, r0, sm0, i),
                                           (g1, s1i, r1, sm1, i + 1)):
                    pltpu.make_async_copy(th.at[gv], rv, sm).wait()
                    pltpu.sync_copy(rv, acc_sh.at[sv], add=True)
                    pltpu.sync_copy(gidx_hbm.at[pl.ds(off(ch + 2), _C)], gv)
                    pltpu.sync_copy(sidx_hbm.at[pl.ds(off(ch + 2), _C)], sv)
                    pltpu.async_copy(th.at[gv], rv, sm)
                return carry

            lax.fori_loop(0, n_it // 2, body, 0)
            pltpu.make_async_copy(th.at[g0], r0, sm0).wait()
            pltpu.make_async_copy(th.at[g1], r1, sm1).wait()
            plsc.subcore_barrier()
            pltpu.sync_copy(acc_sh.at[pl.ds(s * rpt, rpt)], stage_v)
            pltpu.sync_copy(stage_v, oh.at[pl.ds(c * tp + s * rpt, rpt)])

    o0, o1 = k(table[:, :DH], table[:, DH:], gidx, sidx,
               jnp.zeros((tp, DH), jnp.float32))
    return jnp.concatenate([o0, o1], axis=1)


def _sc_scatter_add(values, sidx, tp):
    M = values.shape[0]
    D = values.shape[1]
    per_w = M // 32
    n_it = per_w // _C
    rpt = tp // 16

    @functools.partial(
        pl.kernel, mesh=plsc.VectorSubcoreMesh(**_SC_MESH),
        compiler_params=_SC_PARAMS,
        out_type=jax.ShapeDtypeStruct((2 * tp, D), jnp.float32),
        scratch_types=[pltpu.VMEM((_C,), jnp.int32),
                       pltpu.VMEM((_C, D), jnp.float32),
                       pltpu.VMEM((rpt, D), jnp.float32),
                       pltpu.VMEM_SHARED((tp, D), jnp.float32)],
    )
    def k(val_hbm, sidx_hbm, zero_hbm, out_hbm,
          si_v, rows_v, stage_v, acc_sh):
        c = lax.axis_index("c")
        s = lax.axis_index("s")
        pltpu.sync_copy(zero_hbm.at[pl.ds(s * rpt, rpt)], stage_v)
        pltpu.sync_copy(stage_v, acc_sh.at[pl.ds(s * rpt, rpt)])
        plsc.subcore_barrier()
        base = (c * 16 + s) * per_w

        def body(i, carry):
            off = base + i * _C
            pltpu.sync_copy(sidx_hbm.at[pl.ds(off, _C)], si_v)
            pltpu.sync_copy(val_hbm.at[pl.ds(off, _C)], rows_v)
            pltpu.sync_copy(rows_v, acc_sh.at[si_v], add=True)
            return carry

        lax.fori_loop(0, n_it, body, 0)
        plsc.subcore_barrier()
        pltpu.sync_copy(acc_sh.at[pl.ds(s * rpt, rpt)], stage_v)
        pltpu.sync_copy(stage_v, out_hbm.at[pl.ds(c * tp + s * rpt, rpt)])

    return k(values, sidx, jnp.zeros((tp, D), jnp.float32))


def _sc_gather2_add(tab1, tab2, idx1, idx2):
    M = idx1.shape[0]
    D = tab1.shape[1]
    per_w = M // 32
    n_it = per_w // _C

    @functools.partial(
        pl.kernel, mesh=plsc.VectorSubcoreMesh(**_SC_MESH),
        compiler_params=_SC_PARAMS,
        out_type=jax.ShapeDtypeStruct((M, D), jnp.float32),
        scratch_types=[pltpu.VMEM((_C,), jnp.int32),
                       pltpu.VMEM((_C,), jnp.int32),
                       pltpu.VMEM((_C, D), jnp.float32),
                       pltpu.VMEM((_C, D), jnp.float32),
                       pltpu.SemaphoreType.DMA],
    )
    def k(t1_hbm, t2_hbm, i1_hbm, i2_hbm, out_hbm,
          i1_v, i2_v, r1_v, r2_v, sem):
        wid = lax.axis_index("s") * 2 + lax.axis_index("c")
        base = wid * per_w

        def body(i, carry):
            off = base + i * _C
            pltpu.sync_copy(i1_hbm.at[pl.ds(off, _C)], i1_v)
            pltpu.sync_copy(i2_hbm.at[pl.ds(off, _C)], i2_v)
            pltpu.async_copy(t1_hbm.at[i1_v], r1_v, sem).wait()
            pltpu.async_copy(t2_hbm.at[i2_v], r2_v, sem).wait()

            def add_row(j, carry2):
                r1_v[j, :] = r1_v[j, :] + r2_v[j, :]
                return carry2

            lax.fori_loop(0, _C, add_row, 0)
            pltpu.sync_copy(r1_v, out_hbm.at[pl.ds(off, _C)])
            return carry

        lax.fori_loop(0, n_it, body, 0)

    return k(tab1, tab2, idx1, idx2)


_NB = 256


def _mm_body(x_ref, w_ref, b_ref, o_ref):
    o_ref[...] = jnp.dot(x_ref[...], w_ref[...],
                         preferred_element_type=jnp.float32) + b_ref[...]


def _tc_matmul(x, w, b):
    n, kdim = x.shape
    dout = w.shape[1]
    return pl.pallas_call(
        _mm_body,
        grid=(n // _NB,),
        in_specs=[pl.BlockSpec((_NB, kdim), lambda i: (i, 0)),
                  pl.BlockSpec((kdim, dout), lambda i: (0, 0)),
                  pl.BlockSpec((1, dout), lambda i: (0, 0))],
        out_specs=pl.BlockSpec((_NB, dout), lambda i: (i, 0)),
        out_shape=jax.ShapeDtypeStruct((n, dout), jnp.float32),
    )(x, w, b[None])


def _gconv_post_body(h2_ref, xt_ref, s0_ref, s1_ref, e0_ref, e1_ref,
                     wev_ref, bev_ref, g_ref, b_ref, o_ref):
    sc = e0_ref[...] + e1_ref[...]
    out = (s0_ref[...] + s1_ref[...] + xt_ref[...]
           + sc[:, 0:1] * wev_ref[...] + sc[:, 1:2] * bev_ref[...])
    y = _silu(out)
    mu = jnp.mean(y, axis=1, keepdims=True)
    v = jnp.mean((y - mu) ** 2, axis=1, keepdims=True)
    o_ref[...] = h2_ref[...] + (y - mu) / jnp.sqrt(v + 1e-5) * g_ref[...] + b_ref[...]


def _tc_gconv_post(h2, xt, s0, s1, e0, e1, p):
    n = h2.shape[0]
    rb = lambda i: (i, 0)
    w0 = lambda i: (0, 0)
    return pl.pallas_call(
        _gconv_post_body,
        grid=(n // _NB,),
        in_specs=[pl.BlockSpec((_NB, HID), rb), pl.BlockSpec((_NB, HID), rb),
                  pl.BlockSpec((_NB, HID), rb), pl.BlockSpec((_NB, HID), rb),
                  pl.BlockSpec((_NB, 16), rb), pl.BlockSpec((_NB, 16), rb),
                  pl.BlockSpec((1, HID), w0), pl.BlockSpec((1, HID), w0),
                  pl.BlockSpec((1, HID), w0), pl.BlockSpec((1, HID), w0)],
        out_specs=pl.BlockSpec((_NB, HID), rb),
        out_shape=jax.ShapeDtypeStruct((n, HID), jnp.float32),
    )(h2, xt, s0, s1, e0, e1,
      p['edge']['w'], p['edge']['b'][None], p['ln']['g'][None], p['ln']['b'][None])


def _egnn_tc_body(has_att, tanh_flag,
                  h_ref, pos_ref, g_ref, em_ref,
                  w1a_ref, w1b_ref, w1r_ref, b1_ref, w2_ref, b2_ref,
                  wc1_ref, bc1_ref, wc2r_ref, bc2_ref,
                  watt_ref, batt_ref,
                  n1a_ref, n1b_ref, bn1_ref, wn2_ref, bn2_ref,
                  gln_ref, bln_ref,
                  ho_ref, po_ref):
    EB = _NB * MAX_NB
    h_r = h_ref[...]
    g = g_ref[...]
    hc = g[:, :HID]
    pc = g[:, HID:HID + 16]
    pos_r = pos_ref[...]
    pos_rep = jnp.reshape(
        jnp.broadcast_to(pos_r[:, None, :], (_NB, MAX_NB, 16)), (EB, 16))
    radial = pos_rep - pc
    rn = jnp.sqrt(jnp.sum(radial * radial, axis=1, keepdims=True))
    rn = jnp.maximum(rn, 1e-8)
    t_r = jnp.dot(h_r, w1a_ref[...], preferred_element_type=jnp.float32)
    t_rep = jnp.reshape(
        jnp.broadcast_to(t_r[:, None, :], (_NB, MAX_NB, HID)), (EB, HID))
    x1 = _silu(t_rep + jnp.dot(hc, w1b_ref[...], preferred_element_type=jnp.float32)
               + rn * w1r_ref[...] + b1_ref[...])
    m = _silu(jnp.dot(x1, w2_ref[...], preferred_element_type=jnp.float32)
              + b2_ref[...])
    cmid = _silu(jnp.dot(m, wc1_ref[...], preferred_element_type=jnp.float32)
                 + bc1_ref[...])
    cd = jnp.sum(cmid * wc2r_ref[...], axis=1, keepdims=True) + bc2_ref[:, 0:1]
    if tanh_flag:
        cd = jnp.tanh(cd)
    emv = em_ref[...][:, 0:1]
    cu = cd * (radial / rn) * emv
    cu3 = jnp.reshape(cu, (_NB, MAX_NB, 16))
    cu_sum = jnp.zeros((_NB, 16), jnp.float32)
    for kk in range(MAX_NB):
        cu_sum = cu_sum + cu3[:, kk, :]
    po_ref[...] = pos_r + cu_sum
    if has_att:
        m = m * jax.nn.sigmoid(
            jnp.sum(m * watt_ref[...], axis=1, keepdims=True) + batt_ref[:, 0:1])
    m = m * emv
    m3 = jnp.reshape(m, (_NB, MAX_NB, HID))
    agg = jnp.zeros((_NB, HID), jnp.float32)
    for kk in range(MAX_NB):
        agg = agg + m3[:, kk, :]
    hn = _silu(jnp.dot(h_r, n1a_ref[...], preferred_element_type=jnp.float32)
               + jnp.dot(agg, n1b_ref[...], preferred_element_type=jnp.float32)
               + bn1_ref[...])
    hn = jnp.dot(hn, wn2_ref[...], preferred_element_type=jnp.float32) + bn2_ref[...]
    hnew = h_r + hn
    mu = jnp.mean(hnew, axis=1, keepdims=True)
    v = jnp.mean((hnew - mu) ** 2, axis=1, keepdims=True)
    ho_ref[...] = (hnew - mu) / jnp.sqrt(v + 1e-5) * gln_ref[...] + bln_ref[...]


def _tc_egnn_layer(h, pos16, gathered, em, p, has_att, tanh_flag):
    n = h.shape[0]
    rb = lambda i: (i, 0)
    eb = lambda i: (i, 0)
    w0 = lambda i: (0, 0)
    w1 = p['e1']['w']
    if has_att:
        watt = p['att']['w'].T
        batt = jnp.broadcast_to(p['att']['b'][None], (1, HID))
    else:
        watt = jnp.zeros((1, HID), jnp.float32)
        batt = jnp.zeros((1, HID), jnp.float32)
    nw = p['n1']['w']
    ho, po = pl.pallas_call(
        functools.partial(_egnn_tc_body, has_att, tanh_flag),
        grid=(n // _NB,),
        in_specs=[pl.BlockSpec((_NB, HID), rb),
                  pl.BlockSpec((_NB, 16), rb),
                  pl.BlockSpec((_NB * MAX_NB, HID + 16), eb),
                  pl.BlockSpec((_NB * MAX_NB, 8), eb),
                  pl.BlockSpec((HID, HID), w0), pl.BlockSpec((HID, HID), w0),
                  pl.BlockSpec((1, HID), w0), pl.BlockSpec((1, HID), w0),
                  pl.BlockSpec((HID, HID), w0), pl.BlockSpec((1, HID), w0),
                  pl.BlockSpec((HID, HID), w0), pl.BlockSpec((1, HID), w0),
                  pl.BlockSpec((1, HID), w0), pl.BlockSpec((1, HID), w0),
                  pl.BlockSpec((1, HID), w0), pl.BlockSpec((1, HID), w0),
                  pl.BlockSpec((HID, HID), w0), pl.BlockSpec((HID, HID), w0),
                  pl.BlockSpec((1, HID), w0), pl.BlockSpec((HID, HID), w0),
                  pl.BlockSpec((1, HID), w0),
                  pl.BlockSpec((1, HID), w0), pl.BlockSpec((1, HID), w0)],
        out_specs=[pl.BlockSpec((_NB, HID), rb), pl.BlockSpec((_NB, 16), rb)],
        out_shape=[jax.ShapeDtypeStruct((n, HID), jnp.float32),
                   jax.ShapeDtypeStruct((n, 16), jnp.float32)],
    )(h, pos16, gathered, em,
      w1[:HID], w1[HID:2 * HID], w1[2 * HID:2 * HID + 1], p['e1']['b'][None],
      p['e2']['w'], p['e2']['b'][None],
      p['c1']['w'], p['c1']['b'][None],
      p['c2']['w'].T, jnp.broadcast_to(p['c2']['b'][None], (1, HID)),
      watt, batt,
      nw[:HID], nw[HID:], p['n1']['b'][None],
      p['n2']['w'], p['n2']['b'][None],
      p['ln']['g'][None], p['ln']['b'][None])
    return ho, po


def kernel(x, pos, edge_attr, params, edge_index, batch):
    N = x.shape[0]
    E = edge_index.shape[1]
    NPAD = ((N + _CH) // _CH) * _CH
    EPAD = ((E + 4095) // 4096) * 4096

    idxp, emp = _radius_graph_pallas(pos, batch, NPAD)
    rcol_flat = idxp.reshape(-1)
    em8 = jnp.pad(emp.reshape(-1)[:, None], ((0, 0), (0, 7)))

    xp = jnp.pad(x, ((0, NPAD - N), (0, 0)))
    pos16 = jnp.pad(pos, ((0, NPAD - N), (0, 13)))
    row2 = jnp.pad(edge_index[0], (0, EPAD - E), constant_values=NPAD - 1)
    col2 = jnp.pad(edge_index[1], (0, EPAD - E))
    ea16 = jnp.pad(
        jnp.concatenate([edge_attr, jnp.ones((E, 1), jnp.float32)], axis=1),
        ((0, EPAD - E), (0, 14)))

    h = _tc_matmul(xp, params['embed8']['w'], params['embed8']['b'])

    esum = _sc_scatter_add(ea16, row2, NPAD)
    e0, e1 = esum[:NPAD], esum[NPAD:]
    p2d = jax.tree.map(lambda *xs: jnp.stack(xs), *params['gnn2d'])

    def gstep(carry, p):
        h2 = carry
        xt = _tc_matmul(h2, p['lin']['w'], p['lin']['b'])
        s = _sc_gather_scatter(xt, col2, row2, NPAD)
        return _tc_gconv_post(h2, xt, s[:NPAD], s[NPAD:], e0, e1, p), None

    h2, _ = lax.scan(gstep, h, p2d)

    h3, pc = h, pos16
    for i, p in enumerate(params['egnn']):
        tb = jnp.concatenate([h3, pc], axis=1)
        gathered = _sc_gather(tb, rcol_flat)
        h3, pc = _tc_egnn_layer(h3, pc, gathered, em8, p,
                                has_att=('att' in p),
                                tanh_flag=(i == NUM_LAYERS - 1))

    hf, atom, pos_pred, p1, p2 = _fusion_heads(h2, h3, pc, params)
    bond = _sc_gather2_add(p1, p2, row2, col2)
    return atom[:N, :11], pos_pred[:N, :3], bond[:E, :4]


# speedup vs baseline: 1.0629x; 1.0074x over previous
import functools

import jax
import jax.numpy as jnp
from jax import lax
from jax.experimental import pallas as pl
from jax.experimental.pallas import tpu as pltpu
from jax.experimental.pallas import tpu_sc as plsc

HID = 128
NUM_LAYERS = 4
CUTOFF = 10.0
MAX_NB = 32


def _silu(x):
    return x * jax.nn.sigmoid(x)


def _apply(p, x):
    return x @ p['w'] + p['b']


def _layernorm(p, x):
    m = jnp.mean(x, axis=-1, keepdims=True)
    v = jnp.var(x, axis=-1, keepdims=True)
    return (x - m) / jnp.sqrt(v + 1e-5) * p['g'] + p['b']


def _fusion_body(h2_ref, h3_ref, w1a_ref, w1b_ref, b1_ref, w2_ref, b2_ref,
                 wa_ref, ba_ref, wp_ref, bp_ref, pc_ref,
                 wb1_ref, wb2_ref, bb_ref,
                 hf_ref, atom_ref, pos_ref, p1_ref, p2_ref):
    t = (h2_ref[...] @ w1a_ref[...] + h3_ref[...] @ w1b_ref[...] + b1_ref[...])
    t = _silu(t)
    hf = t @ w2_ref[...] + b2_ref[...]
    hf_ref[...] = hf
    atom_ref[...] = hf @ wa_ref[...] + ba_ref[...]
    pos_ref[...] = pc_ref[...] + hf @ wp_ref[...] + bp_ref[...]
    p1_ref[...] = hf @ wb1_ref[...] + bb_ref[...]
    p2_ref[...] = hf @ wb2_ref[...]


def _fusion_heads(h2, h3, pc16, params):
    n = h2.shape[0]
    blk = 512
    w1 = params['fusion1']['w']
    w1a, w1b = w1[:HID], w1[HID:]
    wa = jnp.pad(params['atom_head']['w'], ((0, 0), (0, 5)))
    ba = jnp.pad(params['atom_head']['b'], ((0, 5)))
    wp = jnp.pad(params['pos_head']['w'], ((0, 0), (0, 13)))
    bp = jnp.pad(params['pos_head']['b'], ((0, 13)))
    wb = params['bond_head']['w']
    wb1 = jnp.pad(wb[:HID], ((0, 0), (0, 12)))
    wb2 = jnp.pad(wb[HID:], ((0, 0), (0, 12)))
    bb = jnp.pad(params['bond_head']['b'], ((0, 12)))
    grid = (n // blk,)
    rb = lambda i: (i, 0)
    full = lambda i: (0, 0)
    hf, atom, pos, p1, p2 = pl.pallas_call(
        _fusion_body,
        grid=grid,
        in_specs=[
            pl.BlockSpec((blk, HID), rb),
            pl.BlockSpec((blk, HID), rb),
            pl.BlockSpec((HID, HID), full),
            pl.BlockSpec((HID, HID), full),
            pl.BlockSpec((1, HID), full),
            pl.BlockSpec((HID, HID), full),
            pl.BlockSpec((1, HID), full),
            pl.BlockSpec((HID, 16), full),
            pl.BlockSpec((1, 16), full),
            pl.BlockSpec((HID, 16), full),
            pl.BlockSpec((1, 16), full),
            pl.BlockSpec((blk, 16), rb),
            pl.BlockSpec((HID, 16), full),
            pl.BlockSpec((HID, 16), full),
            pl.BlockSpec((1, 16), full),
        ],
        out_specs=[
            pl.BlockSpec((blk, HID), rb),
            pl.BlockSpec((blk, 16), rb),
            pl.BlockSpec((blk, 16), rb),
            pl.BlockSpec((blk, 16), rb),
            pl.BlockSpec((blk, 16), rb),
        ],
        out_shape=[
            jax.ShapeDtypeStruct((n, HID), jnp.float32),
            jax.ShapeDtypeStruct((n, 16), jnp.float32),
            jax.ShapeDtypeStruct((n, 16), jnp.float32),
            jax.ShapeDtypeStruct((n, 16), jnp.float32),
            jax.ShapeDtypeStruct((n, 16), jnp.float32),
        ],
    )(h2, h3, w1a, w1b, params['fusion1']['b'][None],
      params['fusion2']['w'], params['fusion2']['b'][None],
      wa, ba[None], wp, bp[None], pc16,
      wb1, wb2, bb[None])
    return hf, atom, pos, p1, p2


_RB = 128
_CH = 128
_FAST = 5
_BIG = 2**30


def _radius_body(pos4_ref, posT_ref, batchr_ref, batchc_ref, idx_ref, em_ref,
                 comp_ref, cmap_ref, full_ref):
    NPAD = posT_ref.shape[1]
    NCH = NPAD // _CH
    INF = jnp.float32(jnp.inf)
    R2 = jnp.float32(CUTOFF * CUTOFF)
    rb = pl.program_id(0)

    pos_r = pos4_ref[...]
    sq_r = jnp.sum(pos_r * pos_r, axis=1, keepdims=True)
    b_r = batchr_ref[...]
    b_lo = jnp.min(b_r)
    b_hi = jnp.max(b_r)
    rowid = rb * _RB + jax.lax.broadcasted_iota(jnp.int32, (_RB, 1), 0)

    bc_row = batchc_ref[...]
    actives = []
    for j in range(NCH):
        bcj = bc_row[:, j * _CH:(j + 1) * _CH]
        actives.append((jnp.max(bcj) >= b_lo) & (jnp.min(bcj) <= b_hi))
    n_active = sum(a.astype(jnp.int32) for a in actives)
    fast = n_active <= _FAST

    def chunk_d2(j):
        pc = posT_ref[:, j * _CH:(j + 1) * _CH]
        sq_c = jnp.sum(pc * pc, axis=0, keepdims=True)
        dot = jnp.dot(pos_r, pc, preferred_element_type=jnp.float32)
        d2 = sq_r + sq_c - 2.0 * dot
        bcj = bc_row[:, j * _CH:(j + 1) * _CH]
        colg = j * _CH + jax.lax.broadcasted_iota(jnp.int32, (1, _CH), 1)
        ok = (b_r == bcj) & (colg != rowid) & (d2 < R2)
        return jnp.where(ok, d2, INF), colg

    @pl.when(fast)
    def _():
        comp_ref[...] = jnp.full(comp_ref.shape, INF, jnp.float32)
        cmap_ref[...] = jnp.full(cmap_ref.shape, -1, jnp.int32)

    slot = jnp.int32(0)
    for j in range(NCH):
        def do_compact(j=j, slot=slot):
            d2c, colg = chunk_d2(j)
            comp_ref[:, pl.ds(slot * _CH, _CH)] = d2c
            cmap_ref[:, pl.ds(slot * _CH, _CH)] = colg
        pl.when(fast & actives[j])(do_compact)
        slot = slot + actives[j].astype(jnp.int32)

    @pl.when(~fast)
    def _():
        full_ref[...] = jnp.full(full_ref.shape, INF, jnp.float32)

    for j in range(NCH):
        def do_full(j=j):
            d2c, _ = chunk_d2(j)
            full_ref[:, j * _CH:(j + 1) * _CH] = d2c
        pl.when((~fast) & actives[j])(do_full)

    colsel_iota = jax.lax.broadcasted_iota(jnp.int32, (1, MAX_NB), 1)

    def extract(ref, colmap):
        def body(j, carry):
            idxacc, emacc = carry
            d2v = ref[...]
            m = jnp.min(d2v, axis=1, keepdims=True)
            valid = m < INF
            sel = d2v == m
            cand = jnp.where(sel, colmap, _BIG)
            idx = jnp.min(cand, axis=1, keepdims=True)
            idx_out = jnp.where(valid, idx, rowid)
            ref[...] = jnp.where(colmap == idx, INF, d2v)
            here = colsel_iota == j
            idxacc = jnp.where(here, idx_out, idxacc)
            emacc = jnp.where(here, valid.astype(jnp.float32), emacc)
            return idxacc, emacc
        return jax.lax.fori_loop(
            0, MAX_NB, body,
            (jnp.zeros((_RB, MAX_NB), jnp.int32),
             jnp.zeros((_RB, MAX_NB), jnp.float32)))

    @pl.when(fast)
    def _():
        ia, ea = extract(comp_ref, cmap_ref[...])
        idx_ref[...] = ia
        em_ref[...] = ea

    @pl.when(~fast)
    def _():
        colmap_full = jax.lax.broadcasted_iota(jnp.int32, (1, NPAD), 1)
        ia, ea = extract(full_ref, colmap_full)
        idx_ref[...] = ia
        em_ref[...] = ea


def _radius_graph_pallas(pos, batch, NPAD):
    N = pos.shape[0]
    pos4 = jnp.pad(pos, ((0, NPAD - N), (0, 1)))
    posT = pos4.T
    batchf = jnp.pad(batch.astype(jnp.float32), (0, NPAD - N),
                     constant_values=99.0)
    batchc = batchf[None, :]
    batchr = batchf[:, None]
    grid = (NPAD // _RB,)
    idx, em = pl.pallas_call(
        _radius_body,
        grid=grid,
        in_specs=[
            pl.BlockSpec((_RB, 4), lambda i: (i, 0)),
            pl.BlockSpec((4, NPAD), lambda i: (0, 0)),
            pl.BlockSpec((_RB, 1), lambda i: (i, 0)),
            pl.BlockSpec((1, NPAD), lambda i: (0, 0)),
        ],
        out_specs=[
            pl.BlockSpec((_RB, MAX_NB), lambda i: (i, 0)),
            pl.BlockSpec((_RB, MAX_NB), lambda i: (i, 0)),
        ],
        out_shape=[
            jax.ShapeDtypeStruct((NPAD, MAX_NB), jnp.int32),
            jax.ShapeDtypeStruct((NPAD, MAX_NB), jnp.float32),
        ],
        scratch_shapes=[
            pltpu.VMEM((_RB, _FAST * _CH), jnp.float32),
            pltpu.VMEM((1, _FAST * _CH), jnp.int32),
            pltpu.VMEM((_RB, NPAD), jnp.float32),
        ],
    )(pos4, posT, batchr, batchc)
    return idx, em


_SC_MESH = dict(core_axis_name="c", subcore_axis_name="s")
_SC_PARAMS = pltpu.CompilerParams(use_tc_tiling_on_sc=False)
_C = 128


def _sc_gather(table, idx):
    M = idx.shape[0]
    D = table.shape[1]
    per_w = M // 32
    n_it = per_w // _C

    @functools.partial(
        pl.kernel, mesh=plsc.VectorSubcoreMesh(**_SC_MESH),
        compiler_params=_SC_PARAMS,
        out_type=jax.ShapeDtypeStruct((M, D), jnp.float32),
        scratch_types=[pltpu.VMEM((_C,), jnp.int32),
                       pltpu.VMEM((_C,), jnp.int32),
                       pltpu.VMEM((_C, D), jnp.float32),
                       pltpu.VMEM((_C, D), jnp.float32),
                       pltpu.SemaphoreType.DMA,
                       pltpu.SemaphoreType.DMA],
    )
    def k(table_hbm, idx_hbm, out_hbm, i0, i1, r0, r1, s0, s1):
        wid = lax.axis_index("s") * 2 + lax.axis_index("c")
        base = wid * per_w

        def off(i):
            return base + jnp.minimum(i, n_it - 1) * _C

        pltpu.sync_copy(idx_hbm.at[pl.ds(base, _C)], i0)
        pltpu.async_copy(table_hbm.at[i0], r0, s0)
        pltpu.sync_copy(idx_hbm.at[pl.ds(base + _C, _C)], i1)
        pltpu.async_copy(table_hbm.at[i1], r1, s1)

        def body(g, carry):
            i = 2 * g
            for iv, rv, sv, ch in ((i0, r0, s0, i), (i1, r1, s1, i + 1)):
                pltpu.make_async_copy(table_hbm.at[iv], rv, sv).wait()
                pltpu.sync_copy(rv, out_hbm.at[pl.ds(base + ch * _C, _C)])
                pltpu.sync_copy(idx_hbm.at[pl.ds(off(ch + 2), _C)], iv)
                pltpu.async_copy(table_hbm.at[iv], rv, sv)
            return carry

        lax.fori_loop(0, n_it // 2, body, 0)
        pltpu.make_async_copy(table_hbm.at[i0], r0, s0).wait()
        pltpu.make_async_copy(table_hbm.at[i1], r1, s1).wait()

    return k(table, idx)


def _sc_gather_scatter(table, gidx, sidx, tp):
    M = gidx.shape[0]
    D = table.shape[1]
    DH = D // 2
    per_w = M // 32
    n_it = per_w // _C
    rpt = tp // 16

    @functools.partial(
        pl.kernel, mesh=plsc.VectorSubcoreMesh(**_SC_MESH),
        compiler_params=_SC_PARAMS,
        out_type=[jax.ShapeDtypeStruct((2 * tp, DH), jnp.float32),
                  jax.ShapeDtypeStruct((2 * tp, DH), jnp.float32)],
        scratch_types=[pltpu.VMEM((_C,), jnp.int32),
                       pltpu.VMEM((_C,), jnp.int32),
                       pltpu.VMEM((_C,), jnp.int32),
                       pltpu.VMEM((_C,), jnp.int32),
                       pltpu.VMEM((_C, DH), jnp.float32),
                       pltpu.VMEM((_C, DH), jnp.float32),
                       pltpu.VMEM((rpt, DH), jnp.float32),
                       pltpu.VMEM_SHARED((tp, DH), jnp.float32),
                       pltpu.SemaphoreType.DMA,
                       pltpu.SemaphoreType.DMA],
    )
    def k(t0_hbm, t1_hbm, gidx_hbm, sidx_hbm, zero_hbm, o0_hbm, o1_hbm,
          g0, g1, s0i, s1i, r0, r1, stage_v, acc_sh, sm0, sm1):
        c = lax.axis_index("c")
        s = lax.axis_index("s")
        base = (c * 16 + s) * per_w

        def off(i):
            return base + jnp.minimum(i, n_it - 1) * _C

        for th, oh in ((t0_hbm, o0_hbm), (t1_hbm, o1_hbm)):
            pltpu.sync_copy(zero_hbm.at[pl.ds(s * rpt, rpt)], stage_v)
            pltpu.sync_copy(stage_v, acc_sh.at[pl.ds(s * rpt, rpt)])
            plsc.subcore_barrier()

            pltpu.sync_copy(gidx_hbm.at[pl.ds(base, _C)], g0)
            pltpu.sync_copy(sidx_hbm.at[pl.ds(base, _C)], s0i)
            pltpu.async_copy(th.at[g0], r0, sm0)
            pltpu.sync_copy(gidx_hbm.at[pl.ds(base + _C, _C)], g1)
            pltpu.sync_copy(sidx_hbm.at[pl.ds(base + _C, _C)], s1i)
            pltpu.async_copy(th.at[g1], r1, sm1)

            def body(g, carry, th=th):
                i = 2 * g
                for gv, sv, rv, sm, ch in ((g0, s0i, r0, sm0, i),
                                           (g1, s1i, r1, sm1, i + 1)):
                    pltpu.make_async_copy(th.at[gv], rv, sm).wait()
                    pltpu.sync_copy(rv, acc_sh.at[sv], add=True)
                    pltpu.sync_copy(gidx_hbm.at[pl.ds(off(ch + 2), _C)], gv)
                    pltpu.sync_copy(sidx_hbm.at[pl.ds(off(ch + 2), _C)], sv)
                    pltpu.async_copy(th.at[gv], rv, sm)
                return carry

            lax.fori_loop(0, n_it // 2, body, 0)
            pltpu.make_async_copy(th.at[g0], r0, sm0).wait()
            pltpu.make_async_copy(th.at[g1], r1, sm1).wait()
            plsc.subcore_barrier()
            pltpu.sync_copy(acc_sh.at[pl.ds(s * rpt, rpt)], stage_v)
            pltpu.sync_copy(stage_v, oh.at[pl.ds(c * tp + s * rpt, rpt)])

    o0, o1 = k(table[:, :DH], table[:, DH:], gidx, sidx,
               jnp.zeros((tp, DH), jnp.float32))
    return jnp.concatenate([o0, o1], axis=1)


def _sc_scatter_add(values, sidx, tp):
    M = values.shape[0]
    D = values.shape[1]
    per_w = M // 32
    n_it = per_w // _C
    rpt = tp // 16

    @functools.partial(
        pl.kernel, mesh=plsc.VectorSubcoreMesh(**_SC_MESH),
        compiler_params=_SC_PARAMS,
        out_type=jax.ShapeDtypeStruct((2 * tp, D), jnp.float32),
        scratch_types=[pltpu.VMEM((_C,), jnp.int32),
                       pltpu.VMEM((_C, D), jnp.float32),
                       pltpu.VMEM((rpt, D), jnp.float32),
                       pltpu.VMEM_SHARED((tp, D), jnp.float32)],
    )
    def k(val_hbm, sidx_hbm, zero_hbm, out_hbm,
          si_v, rows_v, stage_v, acc_sh):
        c = lax.axis_index("c")
        s = lax.axis_index("s")
        pltpu.sync_copy(zero_hbm.at[pl.ds(s * rpt, rpt)], stage_v)
        pltpu.sync_copy(stage_v, acc_sh.at[pl.ds(s * rpt, rpt)])
        plsc.subcore_barrier()
        base = (c * 16 + s) * per_w

        def body(i, carry):
            off = base + i * _C
            pltpu.sync_copy(sidx_hbm.at[pl.ds(off, _C)], si_v)
            pltpu.sync_copy(val_hbm.at[pl.ds(off, _C)], rows_v)
            pltpu.sync_copy(rows_v, acc_sh.at[si_v], add=True)
            return carry

        lax.fori_loop(0, n_it, body, 0)
        plsc.subcore_barrier()
        pltpu.sync_copy(acc_sh.at[pl.ds(s * rpt, rpt)], stage_v)
        pltpu.sync_copy(stage_v, out_hbm.at[pl.ds(c * tp + s * rpt, rpt)])

    return k(values, sidx, jnp.zeros((tp, D), jnp.float32))


def _sc_gather2_add(tab1, tab2, idx1, idx2):
    M = idx1.shape[0]
    D = tab1.shape[1]
    per_w = M // 32
    n_it = per_w // _C

    @functools.partial(
        pl.kernel, mesh=plsc.VectorSubcoreMesh(**_SC_MESH),
        compiler_params=_SC_PARAMS,
        out_type=jax.ShapeDtypeStruct((M, D), jnp.float32),
        scratch_types=[pltpu.VMEM((_C,), jnp.int32),
                       pltpu.VMEM((_C,), jnp.int32),
                       pltpu.VMEM((_C, D), jnp.float32),
                       pltpu.VMEM((_C, D), jnp.float32),
                       pltpu.SemaphoreType.DMA],
    )
    def k(t1_hbm, t2_hbm, i1_hbm, i2_hbm, out_hbm,
          i1_v, i2_v, r1_v, r2_v, sem):
        wid = lax.axis_index("s") * 2 + lax.axis_index("c")
        base = wid * per_w

        def body(i, carry):
            off = base + i * _C
            pltpu.sync_copy(i1_hbm.at[pl.ds(off, _C)], i1_v)
            pltpu.sync_copy(i2_hbm.at[pl.ds(off, _C)], i2_v)
            pltpu.async_copy(t1_hbm.at[i1_v], r1_v, sem).wait()
            pltpu.async_copy(t2_hbm.at[i2_v], r2_v, sem).wait()

            def add_row(j, carry2):
                r1_v[j, :] = r1_v[j, :] + r2_v[j, :]
                return carry2

            lax.fori_loop(0, _C, add_row, 0)
            pltpu.sync_copy(r1_v, out_hbm.at[pl.ds(off, _C)])
            return carry

        lax.fori_loop(0, n_it, body, 0)

    return k(tab1, tab2, idx1, idx2)


_NB = 256


def _mm_body(x_ref, w_ref, b_ref, o_ref):
    o_ref[...] = jnp.dot(x_ref[...], w_ref[...],
                         preferred_element_type=jnp.float32) + b_ref[...]


def _tc_matmul(x, w, b):
    n, kdim = x.shape
    dout = w.shape[1]
    return pl.pallas_call(
        _mm_body,
        grid=(n // _NB,),
        in_specs=[pl.BlockSpec((_NB, kdim), lambda i: (i, 0)),
                  pl.BlockSpec((kdim, dout), lambda i: (0, 0)),
                  pl.BlockSpec((1, dout), lambda i: (0, 0))],
        out_specs=pl.BlockSpec((_NB, dout), lambda i: (i, 0)),
        out_shape=jax.ShapeDtypeStruct((n, dout), jnp.float32),
    )(x, w, b[None])


def _gconv_post_body(h2_ref, xt_ref, s0_ref, s1_ref, e0_ref, e1_ref,
                     wev_ref, bev_ref, g_ref, b_ref, o_ref):
    sc = e0_ref[...] + e1_ref[...]
    out = (s0_ref[...] + s1_ref[...] + xt_ref[...]
           + sc[:, 0:1] * wev_ref[...] + sc[:, 1:2] * bev_ref[...])
    y = _silu(out)
    mu = jnp.mean(y, axis=1, keepdims=True)
    v = jnp.mean((y - mu) ** 2, axis=1, keepdims=True)
    o_ref[...] = h2_ref[...] + (y - mu) / jnp.sqrt(v + 1e-5) * g_ref[...] + b_ref[...]


def _tc_gconv_post(h2, xt, s0, s1, e0, e1, p):
    n = h2.shape[0]
    rb = lambda i: (i, 0)
    w0 = lambda i: (0, 0)
    return pl.pallas_call(
        _gconv_post_body,
        grid=(n // _NB,),
        in_specs=[pl.BlockSpec((_NB, HID), rb), pl.BlockSpec((_NB, HID), rb),
                  pl.BlockSpec((_NB, HID), rb), pl.BlockSpec((_NB, HID), rb),
                  pl.BlockSpec((_NB, 16), rb), pl.BlockSpec((_NB, 16), rb),
                  pl.BlockSpec((1, HID), w0), pl.BlockSpec((1, HID), w0),
                  pl.BlockSpec((1, HID), w0), pl.BlockSpec((1, HID), w0)],
        out_specs=pl.BlockSpec((_NB, HID), rb),
        out_shape=jax.ShapeDtypeStruct((n, HID), jnp.float32),
    )(h2, xt, s0, s1, e0, e1,
      p['edge']['w'], p['edge']['b'][None], p['ln']['g'][None], p['ln']['b'][None])


def _egnn_tc_body(has_att, tanh_flag,
                  h_ref, pos_ref, g_ref, em_ref,
                  w1a_ref, w1b_ref, w1r_ref, b1_ref, w2_ref, b2_ref,
                  wc1_ref, bc1_ref, wc2r_ref, bc2_ref,
                  watt_ref, batt_ref,
                  n1a_ref, n1b_ref, bn1_ref, wn2_ref, bn2_ref,
                  gln_ref, bln_ref,
                  ho_ref, po_ref):
    EB = _NB * MAX_NB
    h_r = h_ref[...]
    g = g_ref[...]
    hc = g[:, :HID]
    pc = g[:, HID:HID + 16]
    pos_r = pos_ref[...]
    pos_rep = jnp.reshape(
        jnp.broadcast_to(pos_r[:, None, :], (_NB, MAX_NB, 16)), (EB, 16))
    radial = pos_rep - pc
    rn = jnp.sqrt(jnp.sum(radial * radial, axis=1, keepdims=True))
    rn = jnp.maximum(rn, 1e-8)
    t_r = jnp.dot(h_r, w1a_ref[...], preferred_element_type=jnp.float32)
    t_rep = jnp.reshape(
        jnp.broadcast_to(t_r[:, None, :], (_NB, MAX_NB, HID)), (EB, HID))
    x1 = _silu(t_rep + jnp.dot(hc, w1b_ref[...], preferred_element_type=jnp.float32)
               + rn * w1r_ref[...] + b1_ref[...])
    m = _silu(jnp.dot(x1, w2_ref[...], preferred_element_type=jnp.float32)
              + b2_ref[...])
    cmid = _silu(jnp.dot(m, wc1_ref[...], preferred_element_type=jnp.float32)
                 + bc1_ref[...])
    cd = jnp.sum(cmid * wc2r_ref[...], axis=1, keepdims=True) + bc2_ref[:, 0:1]
    if tanh_flag:
        cd = jnp.tanh(cd)
    emv = em_ref[...][:, 0:1]
    cu = cd * (radial / rn) * emv
    cu3 = jnp.reshape(cu, (_NB, MAX_NB, 16))
    cu_sum = jnp.zeros((_NB, 16), jnp.float32)
    for kk in range(MAX_NB):
        cu_sum = cu_sum + cu3[:, kk, :]
    po_ref[...] = pos_r + cu_sum
    if has_att:
        m = m * jax.nn.sigmoid(
            jnp.sum(m * watt_ref[...], axis=1, keepdims=True) + batt_ref[:, 0:1])
    m = m * emv
    m3 = jnp.reshape(m, (_NB, MAX_NB, HID))
    agg = jnp.zeros((_NB, HID), jnp.float32)
    for kk in range(MAX_NB):
        agg = agg + m3[:, kk, :]
    hn = _silu(jnp.dot(h_r, n1a_ref[...], preferred_element_type=jnp.float32)
               + jnp.dot(agg, n1b_ref[...], preferred_element_type=jnp.float32)
               + bn1_ref[...])
    hn = jnp.dot(hn, wn2_ref[...], preferred_element_type=jnp.float32) + bn2_ref[...]
    hnew = h_r + hn
    mu = jnp.mean(hnew, axis=1, keepdims=True)
    v = jnp.mean((hnew - mu) ** 2, axis=1, keepdims=True)
    ho_ref[...] = (hnew - mu) / jnp.sqrt(v + 1e-5) * gln_ref[...] + bln_ref[...]


def _tc_egnn_layer(h, pos16, gathered, em, p, has_att, tanh_flag):
    n = h.shape[0]
    rb = lambda i: (i, 0)
    eb = lambda i: (i, 0)
    w0 = lambda i: (0, 0)
    w1 = p['e1']['w']
    if has_att:
        watt = p['att']['w'].T
        batt = jnp.broadcast_to(p['att']['b'][None], (1, HID))
    else:
        watt = jnp.zeros((1, HID), jnp.float32)
        batt = jnp.zeros((1, HID), jnp.float32)
    nw = p['n1']['w']
    ho, po = pl.pallas_call(
        functools.partial(_egnn_tc_body, has_att, tanh_flag),
        grid=(n // _NB,),
        in_specs=[pl.BlockSpec((_NB, HID), rb),
                  pl.BlockSpec((_NB, 16), rb),
                  pl.BlockSpec((_NB * MAX_NB, HID + 16), eb),
                  pl.BlockSpec((_NB * MAX_NB, 8), eb),
                  pl.BlockSpec((HID, HID), w0), pl.BlockSpec((HID, HID), w0),
                  pl.BlockSpec((1, HID), w0), pl.BlockSpec((1, HID), w0),
                  pl.BlockSpec((HID, HID), w0), pl.BlockSpec((1, HID), w0),
                  pl.BlockSpec((HID, HID), w0), pl.BlockSpec((1, HID), w0),
                  pl.BlockSpec((1, HID), w0), pl.BlockSpec((1, HID), w0),
                  pl.BlockSpec((1, HID), w0), pl.BlockSpec((1, HID), w0),
                  pl.BlockSpec((HID, HID), w0), pl.BlockSpec((HID, HID), w0),
                  pl.BlockSpec((1, HID), w0), pl.BlockSpec((HID, HID), w0),
                  pl.BlockSpec((1, HID), w0),
                  pl.BlockSpec((1, HID), w0), pl.BlockSpec((1, HID), w0)],
        out_specs=[pl.BlockSpec((_NB, HID), rb), pl.BlockSpec((_NB, 16), rb)],
        out_shape=[jax.ShapeDtypeStruct((n, HID), jnp.float32),
                   jax.ShapeDtypeStruct((n, 16), jnp.float32)],
    )(h, pos16, gathered, em,
      w1[:HID], w1[HID:2 * HID], w1[2 * HID:2 * HID + 1], p['e1']['b'][None],
      p['e2']['w'], p['e2']['b'][None],
      p['c1']['w'], p['c1']['b'][None],
      p['c2']['w'].T, jnp.broadcast_to(p['c2']['b'][None], (1, HID)),
      watt, batt,
      nw[:HID], nw[HID:], p['n1']['b'][None],
      p['n2']['w'], p['n2']['b'][None],
      p['ln']['g'][None], p['ln']['b'][None])
    return ho, po


def kernel(x, pos, edge_attr, params, edge_index, batch):
    N = x.shape[0]
    E = edge_index.shape[1]
    NPAD = ((N + _CH) // _CH) * _CH
    EPAD = ((E + 4095) // 4096) * 4096

    idxp, emp = _radius_graph_pallas(pos, batch, NPAD)
    rcol_flat = idxp.reshape(-1)
    em8 = jnp.pad(emp.reshape(-1)[:, None], ((0, 0), (0, 7)))

    xp = jnp.pad(x, ((0, NPAD - N), (0, 0)))
    pos16 = jnp.pad(pos, ((0, NPAD - N), (0, 13)))
    row2 = jnp.pad(edge_index[0], (0, EPAD - E), constant_values=NPAD - 1)
    col2 = jnp.pad(edge_index[1], (0, EPAD - E))
    ea16 = jnp.pad(
        jnp.concatenate([edge_attr, jnp.ones((E, 1), jnp.float32)], axis=1),
        ((0, EPAD - E), (0, 14)))

    h = _tc_matmul(xp, params['embed8']['w'], params['embed8']['b'])

    esum = _sc_scatter_add(ea16, row2, NPAD)
    e0, e1 = esum[:NPAD], esum[NPAD:]
    p2d = jax.tree.map(lambda *xs: jnp.stack(xs), *params['gnn2d'])

    def gstep(carry, p):
        h2 = carry
        xt = _tc_matmul(h2, p['lin']['w'], p['lin']['b'])
        s = _sc_gather_scatter(xt, col2, row2, NPAD)
        return _tc_gconv_post(h2, xt, s[:NPAD], s[NPAD:], e0, e1, p), None

    h2, _ = lax.scan(gstep, h, p2d)

    h3, pc = h, pos16
    for i, p in enumerate(params['egnn']):
        tb = jnp.concatenate([h3, pc], axis=1)
        gathered = _sc_gather(tb, rcol_flat)
        h3, pc = _tc_egnn_layer(h3, pc, gathered, em8, p,
                                has_att=('att' in p),
                                tanh_flag=(i == NUM_LAYERS - 1))

    hf, atom, pos_pred, p1, p2 = _fusion_heads(h2, h3, pc, params)
    bond = _sc_gather2_add(p1, p2, row2, col2)
    return atom[:N, :11], pos_pred[:N, :3], bond[:E, :4]


# speedup vs baseline: 1.0746x; 1.0110x over previous
import functools

import jax
import jax.numpy as jnp
from jax import lax
from jax.experimental import pallas as pl
from jax.experimental.pallas import tpu as pltpu
from jax.experimental.pallas import tpu_sc as plsc

HID = 128
NUM_LAYERS = 4
CUTOFF = 10.0
MAX_NB = 32


def _silu(x):
    return x * jax.nn.sigmoid(x)


def _apply(p, x):
    return x @ p['w'] + p['b']


def _layernorm(p, x):
    m = jnp.mean(x, axis=-1, keepdims=True)
    v = jnp.var(x, axis=-1, keepdims=True)
    return (x - m) / jnp.sqrt(v + 1e-5) * p['g'] + p['b']


def _fusion_body(h2_ref, h3_ref, w1a_ref, w1b_ref, b1_ref, w2_ref, b2_ref,
                 wa_ref, ba_ref, wp_ref, bp_ref, pc_ref,
                 wb1_ref, wb2_ref, bb_ref,
                 hf_ref, atom_ref, pos_ref, p1_ref, p2_ref):
    t = (h2_ref[...] @ w1a_ref[...] + h3_ref[...] @ w1b_ref[...] + b1_ref[...])
    t = _silu(t)
    hf = t @ w2_ref[...] + b2_ref[...]
    hf_ref[...] = hf
    atom_ref[...] = hf @ wa_ref[...] + ba_ref[...]
    pos_ref[...] = pc_ref[...] + hf @ wp_ref[...] + bp_ref[...]
    p1_ref[...] = hf @ wb1_ref[...] + bb_ref[...]
    p2_ref[...] = hf @ wb2_ref[...]


def _fusion_heads(h2, h3, pc16, params):
    n = h2.shape[0]
    blk = 512
    w1 = params['fusion1']['w']
    w1a, w1b = w1[:HID], w1[HID:]
    wa = jnp.pad(params['atom_head']['w'], ((0, 0), (0, 5)))
    ba = jnp.pad(params['atom_head']['b'], ((0, 5)))
    wp = jnp.pad(params['pos_head']['w'], ((0, 0), (0, 13)))
    bp = jnp.pad(params['pos_head']['b'], ((0, 13)))
    wb = params['bond_head']['w']
    wb1 = jnp.pad(wb[:HID], ((0, 0), (0, 12)))
    wb2 = jnp.pad(wb[HID:], ((0, 0), (0, 12)))
    bb = jnp.pad(params['bond_head']['b'], ((0, 12)))
    grid = (n // blk,)
    rb = lambda i: (i, 0)
    full = lambda i: (0, 0)
    hf, atom, pos, p1, p2 = pl.pallas_call(
        _fusion_body,
        grid=grid,
        in_specs=[
            pl.BlockSpec((blk, HID), rb),
            pl.BlockSpec((blk, HID), rb),
            pl.BlockSpec((HID, HID), full),
            pl.BlockSpec((HID, HID), full),
            pl.BlockSpec((1, HID), full),
            pl.BlockSpec((HID, HID), full),
            pl.BlockSpec((1, HID), full),
            pl.BlockSpec((HID, 16), full),
            pl.BlockSpec((1, 16), full),
            pl.BlockSpec((HID, 16), full),
            pl.BlockSpec((1, 16), full),
            pl.BlockSpec((blk, 16), rb),
            pl.BlockSpec((HID, 16), full),
            pl.BlockSpec((HID, 16), full),
            pl.BlockSpec((1, 16), full),
        ],
        out_specs=[
            pl.BlockSpec((blk, HID), rb),
            pl.BlockSpec((blk, 16), rb),
            pl.BlockSpec((blk, 16), rb),
            pl.BlockSpec((blk, 16), rb),
            pl.BlockSpec((blk, 16), rb),
        ],
        out_shape=[
            jax.ShapeDtypeStruct((n, HID), jnp.float32),
            jax.ShapeDtypeStruct((n, 16), jnp.float32),
            jax.ShapeDtypeStruct((n, 16), jnp.float32),
            jax.ShapeDtypeStruct((n, 16), jnp.float32),
            jax.ShapeDtypeStruct((n, 16), jnp.float32),
        ],
    )(h2, h3, w1a, w1b, params['fusion1']['b'][None],
      params['fusion2']['w'], params['fusion2']['b'][None],
      wa, ba[None], wp, bp[None], pc16,
      wb1, wb2, bb[None])
    return hf, atom, pos, p1, p2


_RB = 128
_CH = 128
_FAST = 5
_BIG = 2**30


def _radius_body(pos4_ref, posT_ref, batchr_ref, batchc_ref, idx_ref, em_ref,
                 comp_ref, cmap_ref, full_ref):
    NPAD = posT_ref.shape[1]
    NCH = NPAD // _CH
    INF = jnp.float32(jnp.inf)
    R2 = jnp.float32(CUTOFF * CUTOFF)
    rb = pl.program_id(0)

    pos_r = pos4_ref[...]
    sq_r = jnp.sum(pos_r * pos_r, axis=1, keepdims=True)
    b_r = batchr_ref[...]
    b_lo = jnp.min(b_r)
    b_hi = jnp.max(b_r)
    rowid = rb * _RB + jax.lax.broadcasted_iota(jnp.int32, (_RB, 1), 0)

    bc_row = batchc_ref[...]
    actives = []
    for j in range(NCH):
        bcj = bc_row[:, j * _CH:(j + 1) * _CH]
        actives.append((jnp.max(bcj) >= b_lo) & (jnp.min(bcj) <= b_hi))
    n_active = sum(a.astype(jnp.int32) for a in actives)
    fast = n_active <= _FAST

    def chunk_d2(j):
        pc = posT_ref[:, j * _CH:(j + 1) * _CH]
        sq_c = jnp.sum(pc * pc, axis=0, keepdims=True)
        dot = jnp.dot(pos_r, pc, preferred_element_type=jnp.float32)
        d2 = sq_r + sq_c - 2.0 * dot
        bcj = bc_row[:, j * _CH:(j + 1) * _CH]
        colg = j * _CH + jax.lax.broadcasted_iota(jnp.int32, (1, _CH), 1)
        ok = (b_r == bcj) & (colg != rowid) & (d2 < R2)
        return jnp.where(ok, d2, INF), colg

    @pl.when(fast)
    def _():
        comp_ref[...] = jnp.full(comp_ref.shape, INF, jnp.float32)
        cmap_ref[...] = jnp.full(cmap_ref.shape, -1, jnp.int32)

    slot = jnp.int32(0)
    for j in range(NCH):
        def do_compact(j=j, slot=slot):
            d2c, colg = chunk_d2(j)
            comp_ref[:, pl.ds(slot * _CH, _CH)] = d2c
            cmap_ref[:, pl.ds(slot * _CH, _CH)] = colg
        pl.when(fast & actives[j])(do_compact)
        slot = slot + actives[j].astype(jnp.int32)

    @pl.when(~fast)
    def _():
        full_ref[...] = jnp.full(full_ref.shape, INF, jnp.float32)

    for j in range(NCH):
        def do_full(j=j):
            d2c, _ = chunk_d2(j)
            full_ref[:, j * _CH:(j + 1) * _CH] = d2c
        pl.when((~fast) & actives[j])(do_full)

    colsel_iota = jax.lax.broadcasted_iota(jnp.int32, (1, MAX_NB), 1)

    def extract(ref, colmap):
        def body(j, carry):
            idxacc, emacc = carry
            d2v = ref[...]
            m = jnp.min(d2v, axis=1, keepdims=True)
            valid = m < INF
            sel = d2v == m
            cand = jnp.where(sel, colmap, _BIG)
            idx = jnp.min(cand, axis=1, keepdims=True)
            idx_out = jnp.where(valid, idx, rowid)
            ref[...] = jnp.where(colmap == idx, INF, d2v)
            here = colsel_iota == j
            idxacc = jnp.where(here, idx_out, idxacc)
            emacc = jnp.where(here, valid.astype(jnp.float32), emacc)
            return idxacc, emacc
        return jax.lax.fori_loop(
            0, MAX_NB, body,
            (jnp.zeros((_RB, MAX_NB), jnp.int32),
             jnp.zeros((_RB, MAX_NB), jnp.float32)))

    @pl.when(fast)
    def _():
        ia, ea = extract(comp_ref, cmap_ref[...])
        idx_ref[...] = ia
        em_ref[...] = ea

    @pl.when(~fast)
    def _():
        colmap_full = jax.lax.broadcasted_iota(jnp.int32, (1, NPAD), 1)
        ia, ea = extract(full_ref, colmap_full)
        idx_ref[...] = ia
        em_ref[...] = ea


def _radius_graph_pallas(pos, batch, NPAD):
    N = pos.shape[0]
    pos4 = jnp.pad(pos, ((0, NPAD - N), (0, 1)))
    posT = pos4.T
    batchf = jnp.pad(batch.astype(jnp.float32), (0, NPAD - N),
                     constant_values=99.0)
    batchc = batchf[None, :]
    batchr = batchf[:, None]
    grid = (NPAD // _RB,)
    idx, em = pl.pallas_call(
        _radius_body,
        grid=grid,
        in_specs=[
            pl.BlockSpec((_RB, 4), lambda i: (i, 0)),
            pl.BlockSpec((4, NPAD), lambda i: (0, 0)),
            pl.BlockSpec((_RB, 1), lambda i: (i, 0)),
            pl.BlockSpec((1, NPAD), lambda i: (0, 0)),
        ],
        out_specs=[
            pl.BlockSpec((_RB, MAX_NB), lambda i: (i, 0)),
            pl.BlockSpec((_RB, MAX_NB), lambda i: (i, 0)),
        ],
        out_shape=[
            jax.ShapeDtypeStruct((NPAD, MAX_NB), jnp.int32),
            jax.ShapeDtypeStruct((NPAD, MAX_NB), jnp.float32),
        ],
        scratch_shapes=[
            pltpu.VMEM((_RB, _FAST * _CH), jnp.float32),
            pltpu.VMEM((1, _FAST * _CH), jnp.int32),
            pltpu.VMEM((_RB, NPAD), jnp.float32),
        ],
    )(pos4, posT, batchr, batchc)
    return idx, em


_SC_MESH = dict(core_axis_name="c", subcore_axis_name="s")
_SC_PARAMS = pltpu.CompilerParams(use_tc_tiling_on_sc=False)
_C = 128


def _sc_gather(table, idx):
    M = idx.shape[0]
    D = table.shape[1]
    per_w = M // 32
    n_it = per_w // _C

    @functools.partial(
        pl.kernel, mesh=plsc.VectorSubcoreMesh(**_SC_MESH),
        compiler_params=_SC_PARAMS,
        out_type=jax.ShapeDtypeStruct((M, D), jnp.float32),
        scratch_types=[pltpu.VMEM((_C,), jnp.int32),
                       pltpu.VMEM((_C,), jnp.int32),
                       pltpu.VMEM((_C, D), jnp.float32),
                       pltpu.VMEM((_C, D), jnp.float32),
                       pltpu.SemaphoreType.DMA,
                       pltpu.SemaphoreType.DMA],
    )
    def k(table_hbm, idx_hbm, out_hbm, i0, i1, r0, r1, s0, s1):
        wid = lax.axis_index("s") * 2 + lax.axis_index("c")
        base = wid * per_w

        def off(i):
            return base + jnp.minimum(i, n_it - 1) * _C

        pltpu.sync_copy(idx_hbm.at[pl.ds(base, _C)], i0)
        pltpu.async_copy(table_hbm.at[i0], r0, s0)
        pltpu.sync_copy(idx_hbm.at[pl.ds(base + _C, _C)], i1)
        pltpu.async_copy(table_hbm.at[i1], r1, s1)

        def body(g, carry):
            i = 2 * g
            for iv, rv, sv, ch in ((i0, r0, s0, i), (i1, r1, s1, i + 1)):
                pltpu.make_async_copy(table_hbm.at[iv], rv, sv).wait()
                pltpu.sync_copy(rv, out_hbm.at[pl.ds(base + ch * _C, _C)])
                pltpu.sync_copy(idx_hbm.at[pl.ds(off(ch + 2), _C)], iv)
                pltpu.async_copy(table_hbm.at[iv], rv, sv)
            return carry

        lax.fori_loop(0, n_it // 2, body, 0)
        pltpu.make_async_copy(table_hbm.at[i0], r0, s0).wait()
        pltpu.make_async_copy(table_hbm.at[i1], r1, s1).wait()

    return k(table, idx)


def _sc_gather_scatter(table, gidx, sidx, tp):
    M = gidx.shape[0]
    D = table.shape[1]
    DH = D // 2
    per_w = M // 32
    n_it = per_w // _C
    rpt = tp // 16

    @functools.partial(
        pl.kernel, mesh=plsc.VectorSubcoreMesh(**_SC_MESH),
        compiler_params=_SC_PARAMS,
        out_type=[jax.ShapeDtypeStruct((2 * tp, DH), jnp.float32),
                  jax.ShapeDtypeStruct((2 * tp, DH), jnp.float32)],
        scratch_types=[pltpu.VMEM((_C,), jnp.int32),
                       pltpu.VMEM((_C,), jnp.int32),
                       pltpu.VMEM((_C,), jnp.int32),
                       pltpu.VMEM((_C,), jnp.int32),
                       pltpu.VMEM((_C, DH), jnp.float32),
                       pltpu.VMEM((_C, DH), jnp.float32),
                       pltpu.VMEM((rpt, DH), jnp.float32),
                       pltpu.VMEM_SHARED((tp, DH), jnp.float32),
                       pltpu.SemaphoreType.DMA,
                       pltpu.SemaphoreType.DMA],
    )
    def k(t0_hbm, t1_hbm, gidx_hbm, sidx_hbm, zero_hbm, o0_hbm, o1_hbm,
          g0, g1, s0i, s1i, r0, r1, stage_v, acc_sh, sm0, sm1):
        c = lax.axis_index("c")
        s = lax.axis_index("s")
        base = (c * 16 + s) * per_w

        def off(i):
            return base + jnp.minimum(i, n_it - 1) * _C

        for th, oh in ((t0_hbm, o0_hbm), (t1_hbm, o1_hbm)):
            pltpu.sync_copy(zero_hbm.at[pl.ds(s * rpt, rpt)], stage_v)
            pltpu.sync_copy(stage_v, acc_sh.at[pl.ds(s * rpt, rpt)])
            plsc.subcore_barrier()

            pltpu.sync_copy(gidx_hbm.at[pl.ds(base, _C)], g0)
            pltpu.sync_copy(sidx_hbm.at[pl.ds(base, _C)], s0i)
            pltpu.async_copy(th.at[g0], r0, sm0)
            pltpu.sync_copy(gidx_hbm.at[pl.ds(base + _C, _C)], g1)
            pltpu.sync_copy(sidx_hbm.at[pl.ds(base + _C, _C)], s1i)
            pltpu.async_copy(th.at[g1], r1, sm1)

            def body(g, carry, th=th):
                i = 2 * g
                for gv, sv, rv, sm, ch in ((g0, s0i, r0, sm0, i),
                                           (g1, s1i, r1, sm1, i + 1)):
                    pltpu.make_async_copy(th.at[gv], rv, sm).wait()
                    pltpu.sync_copy(rv, acc_sh.at[sv], add=True)
                    pltpu.sync_copy(gidx_hbm.at[pl.ds(off(ch + 2), _C)], gv)
                    pltpu.sync_copy(sidx_hbm.at[pl.ds(off(ch + 2), _C)], sv)
                    pltpu.async_copy(th.at[gv], rv, sm)
                return carry

            lax.fori_loop(0, n_it // 2, body, 0)
            pltpu.make_async_copy(th.at[g0], r0, sm0).wait()
            pltpu.make_async_copy(th.at[g1], r1, sm1).wait()
            plsc.subcore_barrier()
            pltpu.sync_copy(acc_sh.at[pl.ds(s * rpt, rpt)], stage_v)
            pltpu.sync_copy(stage_v, oh.at[pl.ds(c * tp + s * rpt, rpt)])

    o0, o1 = k(table[:, :DH], table[:, DH:], gidx, sidx,
               jnp.zeros((tp, DH), jnp.float32))
    return jnp.concatenate([o0, o1], axis=1)


def _sc_scatter_add(values, sidx, tp):
    M = values.shape[0]
    D = values.shape[1]
    per_w = M // 32
    n_it = per_w // _C
    rpt = tp // 16

    @functools.partial(
        pl.kernel, mesh=plsc.VectorSubcoreMesh(**_SC_MESH),
        compiler_params=_SC_PARAMS,
        out_type=jax.ShapeDtypeStruct((2 * tp, D), jnp.float32),
        scratch_types=[pltpu.VMEM((_C,), jnp.int32),
                       pltpu.VMEM((_C, D), jnp.float32),
                       pltpu.VMEM((rpt, D), jnp.float32),
                       pltpu.VMEM_SHARED((tp, D), jnp.float32)],
    )
    def k(val_hbm, sidx_hbm, zero_hbm, out_hbm,
          si_v, rows_v, stage_v, acc_sh):
        c = lax.axis_index("c")
        s = lax.axis_index("s")
        pltpu.sync_copy(zero_hbm.at[pl.ds(s * rpt, rpt)], stage_v)
        pltpu.sync_copy(stage_v, acc_sh.at[pl.ds(s * rpt, rpt)])
        plsc.subcore_barrier()
        base = (c * 16 + s) * per_w

        def body(i, carry):
            off = base + i * _C
            pltpu.sync_copy(sidx_hbm.at[pl.ds(off, _C)], si_v)
            pltpu.sync_copy(val_hbm.at[pl.ds(off, _C)], rows_v)
            pltpu.sync_copy(rows_v, acc_sh.at[si_v], add=True)
            return carry

        lax.fori_loop(0, n_it, body, 0)
        plsc.subcore_barrier()
        pltpu.sync_copy(acc_sh.at[pl.ds(s * rpt, rpt)], stage_v)
        pltpu.sync_copy(stage_v, out_hbm.at[pl.ds(c * tp + s * rpt, rpt)])

    return k(values, sidx, jnp.zeros((tp, D), jnp.float32))


def _sc_gather2_add(tab1, tab2, idx1, idx2):
    M = idx1.shape[0]
    D = tab1.shape[1]
    per_w = M // 32
    n_it = per_w // _C

    @functools.partial(
        pl.kernel, mesh=plsc.VectorSubcoreMesh(**_SC_MESH),
        compiler_params=_SC_PARAMS,
        out_type=jax.ShapeDtypeStruct((M, D), jnp.float32),
        scratch_types=[pltpu.VMEM((2, _C), jnp.int32),
                       pltpu.VMEM((2, _C), jnp.int32),
                       pltpu.VMEM((2, _C, D), jnp.float32),
                       pltpu.VMEM((2, _C, D), jnp.float32),
                       pltpu.SemaphoreType.DMA,
                       pltpu.SemaphoreType.DMA,
                       pltpu.SemaphoreType.DMA,
                       pltpu.SemaphoreType.DMA],
    )
    def k(t1_hbm, t2_hbm, i1_hbm, i2_hbm, out_hbm,
          i1_v, i2_v, r1_v, r2_v, sa0, sb0, sa1, sb1):
        wid = lax.axis_index("s") * 2 + lax.axis_index("c")
        base = wid * per_w
        sems = ((sa0, sb0), (sa1, sb1))

        def off(i):
            return base + jnp.minimum(i, n_it - 1) * _C

        def fire(b, i):
            sa, sb = sems[b]
            pltpu.sync_copy(i1_hbm.at[pl.ds(off(i), _C)], i1_v.at[b])
            pltpu.sync_copy(i2_hbm.at[pl.ds(off(i), _C)], i2_v.at[b])
            pltpu.async_copy(t1_hbm.at[i1_v.at[b]], r1_v.at[b], sa)
            pltpu.async_copy(t2_hbm.at[i2_v.at[b]], r2_v.at[b], sb)

        def drain(b):
            sa, sb = sems[b]
            pltpu.make_async_copy(t1_hbm.at[i1_v.at[b]], r1_v.at[b], sa).wait()
            pltpu.make_async_copy(t2_hbm.at[i2_v.at[b]], r2_v.at[b], sb).wait()

        fire(0, 0)
        fire(1, 1)

        def body(g, carry):
            i = 2 * g
            for b in (0, 1):
                drain(b)

                def add_row(j, carry2, b=b):
                    r1_v[b, j, :] = r1_v[b, j, :] + r2_v[b, j, :]
                    return carry2

                lax.fori_loop(0, _C, add_row, 0)
                pltpu.sync_copy(r1_v.at[b],
                                out_hbm.at[pl.ds(base + (i + b) * _C, _C)])
                fire(b, i + b + 2)
            return carry

        lax.fori_loop(0, n_it // 2, body, 0)
        drain(0)
        drain(1)

    return k(tab1, tab2, idx1, idx2)


_NB = 256


def _mm_body(x_ref, w_ref, b_ref, o_ref):
    o_ref[...] = jnp.dot(x_ref[...], w_ref[...],
                         preferred_element_type=jnp.float32) + b_ref[...]


def _tc_matmul(x, w, b):
    n, kdim = x.shape
    dout = w.shape[1]
    return pl.pallas_call(
        _mm_body,
        grid=(n // _NB,),
        in_specs=[pl.BlockSpec((_NB, kdim), lambda i: (i, 0)),
                  pl.BlockSpec((kdim, dout), lambda i: (0, 0)),
                  pl.BlockSpec((1, dout), lambda i: (0, 0))],
        out_specs=pl.BlockSpec((_NB, dout), lambda i: (i, 0)),
        out_shape=jax.ShapeDtypeStruct((n, dout), jnp.float32),
    )(x, w, b[None])


def _gconv_post_body(h2_ref, xt_ref, s0_ref, s1_ref, e0_ref, e1_ref,
                     wev_ref, bev_ref, g_ref, b_ref, o_ref):
    sc = e0_ref[...] + e1_ref[...]
    out = (s0_ref[...] + s1_ref[...] + xt_ref[...]
           + sc[:, 0:1] * wev_ref[...] + sc[:, 1:2] * bev_ref[...])
    y = _silu(out)
    mu = jnp.mean(y, axis=1, keepdims=True)
    v = jnp.mean((y - mu) ** 2, axis=1, keepdims=True)
    o_ref[...] = h2_ref[...] + (y - mu) / jnp.sqrt(v + 1e-5) * g_ref[...] + b_ref[...]


def _tc_gconv_post(h2, xt, s0, s1, e0, e1, p):
    n = h2.shape[0]
    rb = lambda i: (i, 0)
    w0 = lambda i: (0, 0)
    return pl.pallas_call(
        _gconv_post_body,
        grid=(n // _NB,),
        in_specs=[pl.BlockSpec((_NB, HID), rb), pl.BlockSpec((_NB, HID), rb),
                  pl.BlockSpec((_NB, HID), rb), pl.BlockSpec((_NB, HID), rb),
                  pl.BlockSpec((_NB, 16), rb), pl.BlockSpec((_NB, 16), rb),
                  pl.BlockSpec((1, HID), w0), pl.BlockSpec((1, HID), w0),
                  pl.BlockSpec((1, HID), w0), pl.BlockSpec((1, HID), w0)],
        out_specs=pl.BlockSpec((_NB, HID), rb),
        out_shape=jax.ShapeDtypeStruct((n, HID), jnp.float32),
    )(h2, xt, s0, s1, e0, e1,
      p['edge']['w'], p['edge']['b'][None], p['ln']['g'][None], p['ln']['b'][None])


def _egnn_tc_body(has_att, tanh_flag,
                  h_ref, pos_ref, g_ref, em_ref,
                  w1a_ref, w1b_ref, w1r_ref, b1_ref, w2_ref, b2_ref,
                  wc1_ref, bc1_ref, wc2r_ref, bc2_ref,
                  watt_ref, batt_ref,
                  n1a_ref, n1b_ref, bn1_ref, wn2_ref, bn2_ref,
                  gln_ref, bln_ref,
                  ho_ref, po_ref):
    EB = _NB * MAX_NB
    h_r = h_ref[...]
    g = g_ref[...]
    hc = g[:, :HID]
    pc = g[:, HID:HID + 16]
    pos_r = pos_ref[...]
    pos_rep = jnp.reshape(
        jnp.broadcast_to(pos_r[:, None, :], (_NB, MAX_NB, 16)), (EB, 16))
    radial = pos_rep - pc
    rn = jnp.sqrt(jnp.sum(radial * radial, axis=1, keepdims=True))
    rn = jnp.maximum(rn, 1e-8)
    t_r = jnp.dot(h_r, w1a_ref[...], preferred_element_type=jnp.float32)
    t_rep = jnp.reshape(
        jnp.broadcast_to(t_r[:, None, :], (_NB, MAX_NB, HID)), (EB, HID))
    x1 = _silu(t_rep + jnp.dot(hc, w1b_ref[...], preferred_element_type=jnp.float32)
               + rn * w1r_ref[...] + b1_ref[...])
    m = _silu(jnp.dot(x1, w2_ref[...], preferred_element_type=jnp.float32)
              + b2_ref[...])
    cmid = _silu(jnp.dot(m, wc1_ref[...], preferred_element_type=jnp.float32)
                 + bc1_ref[...])
    cd = jnp.sum(cmid * wc2r_ref[...], axis=1, keepdims=True) + bc2_ref[:, 0:1]
    if tanh_flag:
        cd = jnp.tanh(cd)
    emv = em_ref[...][:, 0:1]
    cu = cd * (radial / rn) * emv
    cu3 = jnp.reshape(cu, (_NB, MAX_NB, 16))
    cu_sum = jnp.zeros((_NB, 16), jnp.float32)
    for kk in range(MAX_NB):
        cu_sum = cu_sum + cu3[:, kk, :]
    po_ref[...] = pos_r + cu_sum
    if has_att:
        m = m * jax.nn.sigmoid(
            jnp.sum(m * watt_ref[...], axis=1, keepdims=True) + batt_ref[:, 0:1])
    m = m * emv
    m3 = jnp.reshape(m, (_NB, MAX_NB, HID))
    agg = jnp.zeros((_NB, HID), jnp.float32)
    for kk in range(MAX_NB):
        agg = agg + m3[:, kk, :]
    hn = _silu(jnp.dot(h_r, n1a_ref[...], preferred_element_type=jnp.float32)
               + jnp.dot(agg, n1b_ref[...], preferred_element_type=jnp.float32)
               + bn1_ref[...])
    hn = jnp.dot(hn, wn2_ref[...], preferred_element_type=jnp.float32) + bn2_ref[...]
    hnew = h_r + hn
    mu = jnp.mean(hnew, axis=1, keepdims=True)
    v = jnp.mean((hnew - mu) ** 2, axis=1, keepdims=True)
    ho_ref[...] = (hnew - mu) / jnp.sqrt(v + 1e-5) * gln_ref[...] + bln_ref[...]


def _tc_egnn_layer(h, pos16, gathered, em, p, has_att, tanh_flag):
    n = h.shape[0]
    rb = lambda i: (i, 0)
    eb = lambda i: (i, 0)
    w0 = lambda i: (0, 0)
    w1 = p['e1']['w']
    if has_att:
        watt = p['att']['w'].T
        batt = jnp.broadcast_to(p['att']['b'][None], (1, HID))
    else:
        watt = jnp.zeros((1, HID), jnp.float32)
        batt = jnp.zeros((1, HID), jnp.float32)
    nw = p['n1']['w']
    ho, po = pl.pallas_call(
        functools.partial(_egnn_tc_body, has_att, tanh_flag),
        grid=(n // _NB,),
        in_specs=[pl.BlockSpec((_NB, HID), rb),
                  pl.BlockSpec((_NB, 16), rb),
                  pl.BlockSpec((_NB * MAX_NB, HID + 16), eb),
                  pl.BlockSpec((_NB * MAX_NB, 8), eb),
                  pl.BlockSpec((HID, HID), w0), pl.BlockSpec((HID, HID), w0),
                  pl.BlockSpec((1, HID), w0), pl.BlockSpec((1, HID), w0),
                  pl.BlockSpec((HID, HID), w0), pl.BlockSpec((1, HID), w0),
                  pl.BlockSpec((HID, HID), w0), pl.BlockSpec((1, HID), w0),
                  pl.BlockSpec((1, HID), w0), pl.BlockSpec((1, HID), w0),
                  pl.BlockSpec((1, HID), w0), pl.BlockSpec((1, HID), w0),
                  pl.BlockSpec((HID, HID), w0), pl.BlockSpec((HID, HID), w0),
                  pl.BlockSpec((1, HID), w0), pl.BlockSpec((HID, HID), w0),
                  pl.BlockSpec((1, HID), w0),
                  pl.BlockSpec((1, HID), w0), pl.BlockSpec((1, HID), w0)],
        out_specs=[pl.BlockSpec((_NB, HID), rb), pl.BlockSpec((_NB, 16), rb)],
        out_shape=[jax.ShapeDtypeStruct((n, HID), jnp.float32),
                   jax.ShapeDtypeStruct((n, 16), jnp.float32)],
    )(h, pos16, gathered, em,
      w1[:HID], w1[HID:2 * HID], w1[2 * HID:2 * HID + 1], p['e1']['b'][None],
      p['e2']['w'], p['e2']['b'][None],
      p['c1']['w'], p['c1']['b'][None],
      p['c2']['w'].T, jnp.broadcast_to(p['c2']['b'][None], (1, HID)),
      watt, batt,
      nw[:HID], nw[HID:], p['n1']['b'][None],
      p['n2']['w'], p['n2']['b'][None],
      p['ln']['g'][None], p['ln']['b'][None])
    return ho, po


def kernel(x, pos, edge_attr, params, edge_index, batch):
    N = x.shape[0]
    E = edge_index.shape[1]
    NPAD = ((N + _CH) // _CH) * _CH
    EPAD = ((E + 4095) // 4096) * 4096

    idxp, emp = _radius_graph_pallas(pos, batch, NPAD)
    rcol_flat = idxp.reshape(-1)
    em8 = jnp.pad(emp.reshape(-1)[:, None], ((0, 0), (0, 7)))

    xp = jnp.pad(x, ((0, NPAD - N), (0, 0)))
    pos16 = jnp.pad(pos, ((0, NPAD - N), (0, 13)))
    row2 = jnp.pad(edge_index[0], (0, EPAD - E), constant_values=NPAD - 1)
    col2 = jnp.pad(edge_index[1], (0, EPAD - E))
    ea16 = jnp.pad(
        jnp.concatenate([edge_attr, jnp.ones((E, 1), jnp.float32)], axis=1),
        ((0, EPAD - E), (0, 14)))

    h = _tc_matmul(xp, params['embed8']['w'], params['embed8']['b'])

    esum = _sc_scatter_add(ea16, row2, NPAD)
    e0, e1 = esum[:NPAD], esum[NPAD:]
    p2d = jax.tree.map(lambda *xs: jnp.stack(xs), *params['gnn2d'])

    def gstep(carry, p):
        h2 = carry
        xt = _tc_matmul(h2, p['lin']['w'], p['lin']['b'])
        s = _sc_gather_scatter(xt, col2, row2, NPAD)
        return _tc_gconv_post(h2, xt, s[:NPAD], s[NPAD:], e0, e1, p), None

    h2, _ = lax.scan(gstep, h, p2d)

    h3, pc = h, pos16
    for i, p in enumerate(params['egnn']):
        tb = jnp.concatenate([h3, pc], axis=1)
        gathered = _sc_gather(tb, rcol_flat)
        h3, pc = _tc_egnn_layer(h3, pc, gathered, em8, p,
                                has_att=('att' in p),
                                tanh_flag=(i == NUM_LAYERS - 1))

    hf, atom, pos_pred, p1, p2 = _fusion_heads(h2, h3, pc, params)
    bond = _sc_gather2_add(p1, p2, row2, col2)
    return atom[:N, :11], pos_pred[:N, :3], bond[:E, :4]
